# baseline (speedup 1.0000x reference)
import functools

import jax
import jax.numpy as jnp
from jax import lax
from jax.experimental import pallas as pl
from jax.experimental.pallas import tpu as pltpu
from jax.experimental.pallas import tpu_sc as plsc

N_NODES = 10000
N_EDGES = 320000
D_FEAT = 128
D_EDGE = 16
D_INT = 64
D_HID = 64
N_LAYERS = 4
D_SCORE = 64
L_READ = 64
D_MODEL = 4
D_INNER = 8
D_STATE = 32
D_CONV = 4
DT_RANK = 1

NC = 2
NS = 16
NW = NC * NS
EPT = N_EDGES // NW
CH = 40
NCH = EPT // CH
NPAD = 10240
RPT = NPAD // NS

@functools.cache
def _sc_mesh():
    return plsc.VectorSubcoreMesh(core_axis_name="c", subcore_axis_name="s")


def _sigmoid16(x):
    return 1.0 / (1.0 + jnp.exp(-x))


def _scf_body(srcb_hbm, dstb_hbm, b3e_hbm, tsrc_hbm, tdst_hbm, zero_hbm,
              ehat_hbm, acc_hbm, stats_hbm,
              sidx_v, didx_v, b3e_v, srow_v, drow_v, vals_v, stat_v,
              acc_sh, isem1, isem2, sem1, sem2, sem3, wsem):
    cid = lax.axis_index("c")
    sid = lax.axis_index("s")
    wid = sid * NC + cid
    pltpu.sync_copy(zero_hbm.at[pl.ds(sid * RPT, RPT)],
                    acc_sh.at[pl.ds(sid * RPT, RPT)])
    plsc.subcore_barrier()
    ebase = wid * EPT
    rbase = wid * NCH

    def issue_idx(c, slot):
        pltpu.async_copy(srcb_hbm.at[pl.ds(rbase + c, 1)],
                         sidx_v.at[pl.ds(slot, 1)], isem1)
        pltpu.async_copy(dstb_hbm.at[pl.ds(rbase + c, 1)],
                         didx_v.at[pl.ds(slot, 1)], isem2)

    def wait_idx():
        pltpu.make_async_copy(srcb_hbm.at[pl.ds(rbase, 1)],
                              sidx_v.at[pl.ds(0, 1)], isem1).wait()
        pltpu.make_async_copy(dstb_hbm.at[pl.ds(rbase, 1)],
                              didx_v.at[pl.ds(0, 1)], isem2).wait()

    def issue_in(c, p):
        base = ebase + c * CH
        slot = lax.rem(c, 2)
        pltpu.async_copy(b3e_hbm.at[pl.ds(base, CH)], b3e_v.at[p], sem1)
        pltpu.async_copy(tsrc_hbm.at[sidx_v.at[slot]], srow_v.at[p], sem2)
        pltpu.async_copy(tdst_hbm.at[didx_v.at[slot]], drow_v.at[p], sem3)

    def wait_in(p):
        pltpu.make_async_copy(b3e_hbm.at[pl.ds(ebase, CH)], b3e_v.at[p], sem1).wait()
        pltpu.make_async_copy(tsrc_hbm.at[sidx_v.at[0]], srow_v.at[p], sem2).wait()
        pltpu.make_async_copy(tdst_hbm.at[didx_v.at[0]], drow_v.at[p], sem3).wait()

    def wait_w(p):
        pltpu.make_async_copy(b3e_v.at[p], ehat_hbm.at[pl.ds(ebase, CH)], wsem).wait()

    pltpu.sync_copy(srcb_hbm.at[pl.ds(rbase, 1)], sidx_v.at[pl.ds(0, 1)])
    pltpu.sync_copy(dstb_hbm.at[pl.ds(rbase, 1)], didx_v.at[pl.ds(0, 1)])
    issue_in(0, 0)
    issue_idx(1, 1)

    def chunk(c, stats):
        p = lax.rem(c, 2)
        @pl.when(c >= 1)
        def _():
            wait_w(1 - p)
        wait_idx()
        issue_in(jnp.minimum(c + 1, NCH - 1), 1 - p)
        wait_in(p)

        def row(r, st):
            out = []
            for v in range(4):
                j = v * 16
                b3 = b3e_v[p, r, pl.ds(j, 16)]
                b1 = srow_v[p, r, pl.ds(j, 16)]
                a2 = srow_v[p, r, pl.ds(64 + j, 16)]
                b2 = drow_v[p, r, pl.ds(j, 16)]
                eh = b3 + b1 + b2
                sg = _sigmoid16(eh)
                b3e_v[p, r, pl.ds(j, 16)] = eh
                vals_v[p, r, pl.ds(j, 16)] = sg * a2
                vals_v[p, r, pl.ds(64 + j, 16)] = sg
                out.append(st[2 * v] + eh)
                out.append(st[2 * v + 1] + eh * eh)
            return tuple(out)

        stats = lax.fori_loop(0, CH, row, stats)
        base = ebase + c * CH
        pltpu.async_copy(b3e_v.at[p], ehat_hbm.at[pl.ds(base, CH)], wsem)
        pltpu.sync_copy(vals_v.at[p], acc_sh.at[didx_v.at[p]], add=True)
        issue_idx(jnp.minimum(c + 2, NCH - 1), p)
        return stats

    zero16 = jnp.zeros((16,), jnp.float32)
    stats = lax.fori_loop(0, NCH, chunk, tuple(zero16 for _ in range(8)))
    wait_w(lax.rem(NCH - 1, 2))
    wait_in(0)
    wait_idx()
    for v in range(4):
        stat_v[v, :] = stats[2 * v]
        stat_v[4 + v, :] = stats[2 * v + 1]
    pltpu.sync_copy(stat_v, stats_hbm.at[wid])
    plsc.subcore_barrier()
    pltpu.sync_copy(acc_sh.at[pl.ds(sid * RPT, RPT)],
                    acc_hbm.at[pl.ds(cid * NPAD + sid * RPT, RPT)])


def _sc_pass_f(srcb, dstb, b3e, tsrc, tdst, zeros_n):
    fn = pl.kernel(
        _scf_body,
        out_type=[
            jax.ShapeDtypeStruct((N_EDGES, D_HID), jnp.float32),
            jax.ShapeDtypeStruct((NC * NPAD, 128), jnp.float32),
            jax.ShapeDtypeStruct((NW, 8, 16), jnp.float32),
        ],
        mesh=_sc_mesh(),
        scratch_types=[
            pltpu.VMEM((2, CH), jnp.int32),
            pltpu.VMEM((2, CH), jnp.int32),
            pltpu.VMEM((2, CH, D_HID), jnp.float32),
            pltpu.VMEM((2, CH, 128), jnp.float32),
            pltpu.VMEM((2, CH, 128), jnp.float32),
            pltpu.VMEM((2, CH, 128), jnp.float32),
            pltpu.VMEM((8, 16), jnp.float32),
            pltpu.VMEM_SHARED((NPAD, 128), jnp.float32),
            pltpu.SemaphoreType.DMA,
            pltpu.SemaphoreType.DMA,
            pltpu.SemaphoreType.DMA,
            pltpu.SemaphoreType.DMA,
            pltpu.SemaphoreType.DMA,
            pltpu.SemaphoreType.DMA,
        ],
    )
    return fn(srcb, dstb, b3e, tsrc, tdst, zeros_n)


def _scb_body(srcb_hbm, dstb_hbm, ehat_hbm, tdst_hbm, zero_hbm,
              acc_hbm,
              sidx_v, didx_v, ehat_v, arow_v,
              acc_sh, isem1, isem2, sem1, sem2):
    cid = lax.axis_index("c")
    sid = lax.axis_index("s")
    wid = sid * NC + cid
    pltpu.sync_copy(zero_hbm.at[pl.ds(sid * RPT, RPT)],
                    acc_sh.at[pl.ds(sid * RPT, RPT)])
    plsc.subcore_barrier()
    ebase = wid * EPT
    rbase = wid * NCH

    def issue_idx(c, slot):
        pltpu.async_copy(srcb_hbm.at[pl.ds(rbase + c, 1)],
                         sidx_v.at[pl.ds(slot, 1)], isem1)
        pltpu.async_copy(dstb_hbm.at[pl.ds(rbase + c, 1)],
                         didx_v.at[pl.ds(slot, 1)], isem2)

    def wait_idx():
        pltpu.make_async_copy(srcb_hbm.at[pl.ds(rbase, 1)],
                              sidx_v.at[pl.ds(0, 1)], isem1).wait()
        pltpu.make_async_copy(dstb_hbm.at[pl.ds(rbase, 1)],
                              didx_v.at[pl.ds(0, 1)], isem2).wait()

    def issue_in(c, p):
        base = ebase + c * CH
        slot = lax.rem(c, 2)
        pltpu.async_copy(ehat_hbm.at[pl.ds(base, CH)], ehat_v.at[p], sem1)
        pltpu.async_copy(tdst_hbm.at[didx_v.at[slot]], arow_v.at[p], sem2)

    def wait_in(p):
        pltpu.make_async_copy(ehat_hbm.at[pl.ds(ebase, CH)], ehat_v.at[p], sem1).wait()
        pltpu.make_async_copy(tdst_hbm.at[didx_v.at[0]], arow_v.at[p], sem2).wait()

    pltpu.sync_copy(srcb_hbm.at[pl.ds(rbase, 1)], sidx_v.at[pl.ds(0, 1)])
    pltpu.sync_copy(dstb_hbm.at[pl.ds(rbase, 1)], didx_v.at[pl.ds(0, 1)])
    issue_in(0, 0)
    issue_idx(1, 1)

    def chunk(c, carry):
        p = lax.rem(c, 2)
        wait_idx()
        issue_in(jnp.minimum(c + 1, NCH - 1), 1 - p)
        wait_in(p)

        def row(r, cr):
            for v in range(4):
                j = v * 16
                eh = ehat_v[p, r, pl.ds(j, 16)]
                a3 = arow_v[p, r, pl.ds(64 + j, 16)]
                sg = _sigmoid16(eh)
                arow_v[p, r, pl.ds(j, 16)] = sg * a3
                arow_v[p, r, pl.ds(64 + j, 16)] = sg
            return cr

        lax.fori_loop(0, CH, row, 0)
        pltpu.sync_copy(arow_v.at[p], acc_sh.at[sidx_v.at[p]], add=True)
        issue_idx(jnp.minimum(c + 2, NCH - 1), p)
        return carry

    lax.fori_loop(0, NCH, chunk, 0)
    wait_in(0)
    wait_idx()
    plsc.subcore_barrier()
    pltpu.sync_copy(acc_sh.at[pl.ds(sid * RPT, RPT)],
                    acc_hbm.at[pl.ds(cid * NPAD + sid * RPT, RPT)])


def _sc_pass_b(srcb, dstb, ehat, tdst, zeros_n):
    fn = pl.kernel(
        _scb_body,
        out_type=[
            jax.ShapeDtypeStruct((NC * NPAD, 128), jnp.float32),
        ],
        mesh=_sc_mesh(),
        scratch_types=[
            pltpu.VMEM((2, CH), jnp.int32),
            pltpu.VMEM((2, CH), jnp.int32),
            pltpu.VMEM((2, CH, D_HID), jnp.float32),
            pltpu.VMEM((2, CH, 128), jnp.float32),
            pltpu.VMEM_SHARED((NPAD, 128), jnp.float32),
            pltpu.SemaphoreType.DMA,
            pltpu.SemaphoreType.DMA,
            pltpu.SemaphoreType.DMA,
            pltpu.SemaphoreType.DMA,
        ],
    )
    return fn(srcb, dstb, ehat, tdst, zeros_n)[0]


def _scg_body(src_hbm, dst_hbm, pp_hbm,
              pre_hbm,
              sidx_v, didx_v, ps_v, pd_v, out_v, sem1, sem2):
    cid = lax.axis_index("c")
    sid = lax.axis_index("s")
    wid = sid * NC + cid
    ebase = wid * EPT

    def chunk(c, carry):
        base = ebase + c * CH
        pltpu.sync_copy(src_hbm.at[pl.ds(base, CH)], sidx_v)
        pltpu.sync_copy(dst_hbm.at[pl.ds(base, CH)], didx_v)
        cp1 = pltpu.async_copy(pp_hbm.at[sidx_v], ps_v, sem1)
        cp2 = pltpu.async_copy(pp_hbm.at[didx_v], pd_v, sem2)
        cp1.wait()
        cp2.wait()

        def row(r, cr):
            for v in range(4):
                j = v * 16
                out_v[r, pl.ds(j, 16)] = (ps_v[r, pl.ds(j, 16)]
                                          + pd_v[r, pl.ds(64 + j, 16)])
            return cr

        lax.fori_loop(0, CH, row, 0)
        pltpu.sync_copy(out_v, pre_hbm.at[pl.ds(base, CH)])
        return carry

    lax.fori_loop(0, NCH, chunk, 0)


def _sc_gather_pre(src, dst, pp):
    fn = pl.kernel(
        _scg_body,
        out_type=[jax.ShapeDtypeStruct((N_EDGES, D_HID), jnp.float32)],
        mesh=_sc_mesh(),
        scratch_types=[
            pltpu.VMEM((CH,), jnp.int32),
            pltpu.VMEM((CH,), jnp.int32),
            pltpu.VMEM((CH, 128), jnp.float32),
            pltpu.VMEM((CH, 128), jnp.float32),
            pltpu.VMEM((CH, D_HID), jnp.float32),
            pltpu.SemaphoreType.DMA,
            pltpu.SemaphoreType.DMA,
        ],
    )
    return fn(src, dst, pp)[0]


def _node_enc_body(x_ref, w1_ref, b1_ref, w2_ref, b2_ref, wn_ref, bn_ref,
                   h_ref, tsrc_ref, tdst_ref):
    h = jnp.maximum(x_ref[...] @ w1_ref[...] + b1_ref[...], 0.0)
    h = h @ w2_ref[...] + b2_ref[...]
    h_ref[...] = h
    proj = h @ wn_ref[...] + bn_ref[...]
    tsrc_ref[...] = proj[:, :128]
    tdst_ref[...] = proj[:, 128:256]


def _node_enc(x, w1, b1, w2, b2, wn, bn):
    return pl.pallas_call(
        _node_enc_body,
        out_shape=[
            jax.ShapeDtypeStruct((N_NODES, D_HID), jnp.float32),
            jax.ShapeDtypeStruct((N_NODES, 128), jnp.float32),
            jax.ShapeDtypeStruct((N_NODES, 128), jnp.float32),
        ],
    )(x, w1, b1, w2, b2, wn, bn)


_EBLK = 6400
_NEB = N_EDGES // _EBLK


def _edge_enc_body(e_ref, w1_ref, b1_ref, w2_ref, b2_ref, w3_ref, b3_ref,
                   e0_ref, b3e_ref):
    e = jnp.maximum(e_ref[...] @ w1_ref[...] + b1_ref[...], 0.0)
    e = e @ w2_ref[...] + b2_ref[...]
    e0_ref[...] = e
    b3e_ref[...] = e @ w3_ref[...] + b3_ref[...]


def _edge_enc(e, w1, b1, w2, b2, w3, b3):
    blk = lambda i: (i, 0)
    cst = lambda i: (0, 0)
    return pl.pallas_call(
        _edge_enc_body,
        grid=(_NEB,),
        in_specs=[
            pl.BlockSpec((_EBLK, D_EDGE), blk),
            pl.BlockSpec((D_EDGE, D_INT), cst),
            pl.BlockSpec((1, D_INT), cst),
            pl.BlockSpec((D_INT, D_HID), cst),
            pl.BlockSpec((1, D_HID), cst),
            pl.BlockSpec((D_HID, D_HID), cst),
            pl.BlockSpec((1, D_HID), cst),
        ],
        out_specs=[
            pl.BlockSpec((_EBLK, D_HID), blk),
            pl.BlockSpec((_EBLK, D_HID), blk),
        ],
        out_shape=[
            jax.ShapeDtypeStruct((N_EDGES, D_HID), jnp.float32),
            jax.ShapeDtypeStruct((N_EDGES, D_HID), jnp.float32),
        ],
    )(e, w1, b1, w2, b2, w3, b3)


def _node_upd_body(h_ref, a1w_ref, a1b_ref, accf_ref, accb_ref, stats_ref,
                   bnh_ref, bne_ref, wn_ref, bn_ref,
                   h2_ref, ss_ref, tsrc_ref, tdst_ref):
    h = h_ref[...]
    a1h = h @ a1w_ref[...] + a1b_ref[...]
    accf = accf_ref[...]
    accb = accb_ref[...]
    num_f = accf[:N_NODES, :64] + accf[NPAD:NPAD + N_NODES, :64]
    den_f = accf[:N_NODES, 64:] + accf[NPAD:NPAD + N_NODES, 64:]
    num_b = accb[:N_NODES, :64] + accb[NPAD:NPAD + N_NODES, :64]
    den_b = accb[:N_NODES, 64:] + accb[NPAD:NPAD + N_NODES, 64:]
    tmp = a1h + num_f / (den_f + 1e-6) + num_b / (den_b + 1e-6)
    mu = jnp.mean(tmp, axis=0, keepdims=True)
    var = jnp.mean((tmp - mu) ** 2, axis=0, keepdims=True)
    bnh = bnh_ref[...]
    hn = (tmp - mu) / jnp.sqrt(var + 1e-5) * bnh[0:1, :] + bnh[1:2, :]
    h2 = h + jnp.maximum(hn, 0.0)
    h2_ref[...] = h2
    st = jnp.sum(stats_ref[...], axis=0)
    mu_e = st[:64] / N_EDGES
    var_e = st[64:] / N_EDGES - mu_e * mu_e
    bne = bne_ref[...]
    scale = bne[0, :] / jnp.sqrt(var_e + 1e-5)
    shift = bne[1, :] - mu_e * scale
    ss_ref[...] = jnp.concatenate(
        [scale[None, :], shift[None, :], jnp.zeros((6, D_HID), jnp.float32)],
        axis=0)
    proj = h2 @ wn_ref[...] + bn_ref[...]
    tsrc_ref[...] = proj[:, :128]
    tdst_ref[...] = proj[:, 128:256]


def _node_upd(h, a1w, a1b, accf, accb, stats, bnh, bne, wn, bn):
    return pl.pallas_call(
        _node_upd_body,
        out_shape=[
            jax.ShapeDtypeStruct((N_NODES, D_HID), jnp.float32),
            jax.ShapeDtypeStruct((8, D_HID), jnp.float32),
            jax.ShapeDtypeStruct((N_NODES, 128), jnp.float32),
            jax.ShapeDtypeStruct((N_NODES, 128), jnp.float32),
        ],
    )(h, a1w, a1b, accf, accb, stats, bnh, bne, wn, bn)


def _node_fin_body(h_ref, a1w_ref, a1b_ref, accf_ref, accb_ref, stats_ref,
                   bnh_ref, bne_ref, x2_ref, ws_ref, wd_ref,
                   ss_ref, pp_ref):
    h = h_ref[...]
    a1h = h @ a1w_ref[...] + a1b_ref[...]
    accf = accf_ref[...]
    accb = accb_ref[...]
    num_f = accf[:N_NODES, :64] + accf[NPAD:NPAD + N_NODES, :64]
    den_f = accf[:N_NODES, 64:] + accf[NPAD:NPAD + N_NODES, 64:]
    num_b = accb[:N_NODES, :64] + accb[NPAD:NPAD + N_NODES, :64]
    den_b = accb[:N_NODES, 64:] + accb[NPAD:NPAD + N_NODES, 64:]
    tmp = a1h + num_f / (den_f + 1e-6) + num_b / (den_b + 1e-6)
    mu = jnp.mean(tmp, axis=0, keepdims=True)
    var = jnp.mean((tmp - mu) ** 2, axis=0, keepdims=True)
    bnh = bnh_ref[...]
    hn = (tmp - mu) / jnp.sqrt(var + 1e-5) * bnh[0:1, :] + bnh[1:2, :]
    hf = h + jnp.maximum(hn, 0.0) + x2_ref[...]
    st = jnp.sum(stats_ref[...], axis=0)
    mu_e = st[:64] / N_EDGES
    var_e = st[64:] / N_EDGES - mu_e * mu_e
    bne = bne_ref[...]
    scale = bne[0, :] / jnp.sqrt(var_e + 1e-5)
    shift = bne[1, :] - mu_e * scale
    ss_ref[...] = jnp.concatenate(
        [scale[None, :], shift[None, :], jnp.zeros((6, D_HID), jnp.float32)],
        axis=0)
    pp_ref[...] = jnp.concatenate([hf @ ws_ref[...], hf @ wd_ref[...]],
                                  axis=1)


def _node_fin(h, a1w, a1b, accf, accb, stats, bnh, bne, x2, ws, wd):
    return pl.pallas_call(
        _node_fin_body,
        out_shape=[
            jax.ShapeDtypeStruct((8, D_HID), jnp.float32),
            jax.ShapeDtypeStruct((N_NODES, 128), jnp.float32),
        ],
    )(h, a1w, a1b, accf, accb, stats, bnh, bne, x2, ws, wd)


def _edge_apply_body(e_ref, ehat_ref, ss_ref, w_ref, b_ref,
                     e2_ref, b3e_ref):
    ss = ss_ref[...]
    en = e_ref[...] + jnp.maximum(ehat_ref[...] * ss[0:1, :] + ss[1:2, :], 0.0)
    e2_ref[...] = en
    b3e_ref[...] = en @ w_ref[...] + b_ref[...]


def _edge_apply(e, ehat, ss, w, b):
    blk = lambda i: (i, 0)
    cst = lambda i: (0, 0)
    return pl.pallas_call(
        _edge_apply_body,
        grid=(_NEB,),
        in_specs=[
            pl.BlockSpec((_EBLK, D_HID), blk),
            pl.BlockSpec((_EBLK, D_HID), blk),
            pl.BlockSpec((8, D_HID), cst),
            pl.BlockSpec((D_HID, D_HID), cst),
            pl.BlockSpec((1, D_HID), cst),
        ],
        out_specs=[
            pl.BlockSpec((_EBLK, D_HID), blk),
            pl.BlockSpec((_EBLK, D_HID), blk),
        ],
        out_shape=[
            jax.ShapeDtypeStruct((N_EDGES, D_HID), jnp.float32),
            jax.ShapeDtypeStruct((N_EDGES, D_HID), jnp.float32),
        ],
    )(e, ehat, ss, w, b)


def _pred_body(pre_ref, e_ref, ehat_ref, ss_ref, w1e_ref, b1_ref,
               w2_ref, b2_ref, out_ref):
    ss = ss_ref[...]
    e4 = e_ref[...] + jnp.maximum(ehat_ref[...] * ss[0:1, :] + ss[1:2, :], 0.0)
    hcat = pre_ref[...] + e4 @ w1e_ref[...] + b1_ref[...]
    hcat = jnp.maximum(hcat, 0.0)
    out_ref[...] = hcat @ w2_ref[...] + b2_ref[...]


def _predictor(pre, e3, ehat4, ss, w1e, b1, w2, b2):
    blk = lambda i: (i, 0)
    cst = lambda i: (0, 0)
    return pl.pallas_call(
        _pred_body,
        grid=(_NEB,),
        in_specs=[
            pl.BlockSpec((_EBLK, D_HID), blk),
            pl.BlockSpec((_EBLK, D_HID), blk),
            pl.BlockSpec((_EBLK, D_HID), blk),
            pl.BlockSpec((8, D_HID), cst),
            pl.BlockSpec((D_HID, D_SCORE), cst),
            pl.BlockSpec((1, D_SCORE), cst),
            pl.BlockSpec((D_SCORE, 1), cst),
            pl.BlockSpec((1, 1), cst),
        ],
        out_specs=pl.BlockSpec((_EBLK, 1), blk),
        out_shape=jax.ShapeDtypeStruct((N_EDGES, 1), jnp.float32),
    )(pre, e3, ehat4, ss, w1e, b1, w2, b2)


_MBLK = 400
_NMB = N_NODES // _MBLK


def _mamba_body(rd_ref, rl_ref, wx_ref, wz_ref, wc_ref, cb_ref,
                mdt_ref, dtb_ref, wbb_ref, wcb_ref, k8_ref, k8t_ref,
                af_ref, df_ref, esel_ref, fsel_ref,
                wo_ref, wb2_ref, bb2_ref, x2_ref, ys_ref):
    rd = rd_ref[...]
    xm = rd @ wx_ref[...]
    z = rd @ wz_ref[...]
    xc = xm @ wc_ref[...] + cb_ref[...]
    xc = xc * _sigmoid16(xc)
    k8 = k8_ref[...]
    af = af_ref[...]
    h = jnp.zeros((_MBLK, 256), jnp.float32)
    for t in range(L_READ):
        xct = xc[:, t * 8:(t + 1) * 8]
        dpre = xct @ mdt_ref[...] + dtb_ref[...]
        dt = jnp.maximum(dpre, 0.0) + jnp.log1p(jnp.exp(-jnp.abs(dpre)))
        d_bc = dt @ k8
        b_bc = xct @ wbb_ref[...]
        c_bc = xct @ wcb_ref[...]
        u_bc = xct @ k8
        dA = jnp.exp(d_bc * af)
        h = dA * h + d_bc * b_bc * u_bc
        yt = (h * c_bc) @ k8t_ref[...]
        ys_ref[:, t * 8:(t + 1) * 8] = yt
    y = ys_ref[...] + xc * df_ref[...]
    y = y * (z * _sigmoid16(z))
    idx = jnp.clip(rl_ref[0, 0, :] - 1, 0, L_READ - 1)
    tmask = (jax.lax.broadcasted_iota(jnp.int32, (_MBLK, L_READ), 1)
             == idx[:, None]).astype(jnp.float32)
    msel = tmask @ esel_ref[...]
    ylast = (y * msel) @ fsel_ref[...]
    out4 = ylast @ wo_ref[...]
    x2_ref[...] = out4 @ wb2_ref[...] + bb2_ref[...]


def _mamba(rd_flat, rl3, m, base_w, base_b):
    inw = m['in_proj_w']
    wx = jnp.zeros((256, 512), jnp.float32)
    wz = jnp.zeros((256, 512), jnp.float32)
    t_i = jnp.arange(L_READ)
    for mm in range(D_MODEL):
        for d in range(D_INNER):
            wx = wx.at[t_i * 4 + mm, t_i * 8 + d].set(inw[d, mm])
            wz = wz.at[t_i * 4 + mm, t_i * 8 + d].set(inw[D_INNER + d, mm])
    wc = jnp.zeros((512, 512), jnp.float32)
    for k in range(D_CONV):
        tt = jnp.arange(D_CONV - 1 - k, L_READ)
        for d in range(D_INNER):
            wc = wc.at[(tt - (D_CONV - 1 - k)) * 8 + d, tt * 8 + d].set(
                m['conv_w'][d, 0, k])
    cb = jnp.tile(m['conv_b'], (L_READ,))[None, :]
    mdt = m['x_proj_w'][:DT_RANK, :].T @ m['dt_proj_w'].T
    dtb = m['dt_proj_b'][None, :]
    k8 = jnp.zeros((8, 256), jnp.float32)
    d_i = jnp.arange(D_INNER)
    s_i = jnp.arange(D_STATE)
    for s in range(D_STATE):
        k8 = k8.at[d_i, d_i * 32 + s].set(1.0)
    k32 = jnp.zeros((32, 256), jnp.float32)
    for d in range(D_INNER):
        k32 = k32.at[s_i, d * 32 + s_i].set(1.0)
    xpb = m['x_proj_w'][DT_RANK:DT_RANK + D_STATE, :]
    xpc = m['x_proj_w'][DT_RANK + D_STATE:, :]
    wbb = xpb.T @ k32
    wcb = xpc.T @ k32
    af = (-jnp.exp(m['A_log'])).reshape(-1)[None, :]
    df = jnp.tile(m['D'], (L_READ,))[None, :]
    esel = jnp.zeros((L_READ, 512), jnp.float32)
    fsel = jnp.zeros((512, 8), jnp.float32)
    for d in range(D_INNER):
        esel = esel.at[t_i, t_i * 8 + d].set(1.0)
        fsel = fsel.at[t_i * 8 + d, d].set(1.0)
    wo = m['out_proj_w'].T
    wb2 = base_w.T
    bb2 = base_b[None, :]

    blk = lambda i: (i, 0)
    cst = lambda i: (0, 0)
    return pl.pallas_call(
        _mamba_body,
        grid=(_NMB,),
        in_specs=[
            pl.BlockSpec((_MBLK, 256), blk),
            pl.BlockSpec((1, 1, _MBLK), lambda i: (i, 0, 0)),
            pl.BlockSpec((256, 512), cst),
            pl.BlockSpec((256, 512), cst),
            pl.BlockSpec((512, 512), cst),
            pl.BlockSpec((1, 512), cst),
            pl.BlockSpec((8, 8), cst),
            pl.BlockSpec((1, 8), cst),
            pl.BlockSpec((8, 256), cst),
            pl.BlockSpec((8, 256), cst),
            pl.BlockSpec((8, 256), cst),
            pl.BlockSpec((256, 8), cst),
            pl.BlockSpec((1, 256), cst),
            pl.BlockSpec((1, 512), cst),
            pl.BlockSpec((L_READ, 512), cst),
            pl.BlockSpec((512, 8), cst),
            pl.BlockSpec((8, 4), cst),
            pl.BlockSpec((4, D_HID), cst),
            pl.BlockSpec((1, D_HID), cst),
        ],
        out_specs=pl.BlockSpec((_MBLK, D_HID), blk),
        out_shape=jax.ShapeDtypeStruct((N_NODES, D_HID), jnp.float32),
        scratch_shapes=[pltpu.VMEM((_MBLK, 512), jnp.float32)],
    )(rd_flat, rl3, wx, wz, wc, cb, mdt, dtb, wbb, wcb, k8, k8.T,
      af, df, esel, fsel, wo, wb2, bb2)


def _pack_node_w(p):
    wn = jnp.concatenate(
        [p['B1_w'].T, p['A2_w'].T, p['B2_w'].T, p['A3_w'].T], axis=1)
    bn = jnp.concatenate(
        [p['B1_b'], p['A2_b'], p['B2_b'], p['A3_b']])[None, :]
    return wn, bn


def kernel(x, e, edge_index, read_data, read_length, params):
    src = edge_index[0]
    dst = edge_index[1]
    srcb = src.reshape(NW * NCH, CH)
    dstb = dst.reshape(NW * NCH, CH)
    p = params
    gnn = p['gnn']
    zeros_n = jnp.zeros((NPAD, 128), jnp.float32)

    wn1, bn1 = _pack_node_w(gnn[0])
    h, tsrc, tdst = _node_enc(
        x, p['l1n_w'].T, p['l1n_b'][None, :], p['l2n_w'].T, p['l2n_b'][None, :],
        wn1, bn1)
    e_cur, b3e = _edge_enc(
        e, p['l1e_w'].T, p['l1e_b'][None, :], p['l2e_w'].T, p['l2e_b'][None, :],
        gnn[0]['B3_w'].T, gnn[0]['B3_b'][None, :])

    rd_flat = read_data.reshape(N_NODES, L_READ * D_MODEL)
    rl3 = read_length.reshape(_NMB, 1, _MBLK)
    x2 = _mamba(rd_flat, rl3, p['mamba'], p['base_w'], p['base_b'])

    ehat = None
    for li in range(N_LAYERS):
        lp = gnn[li]
        ehat, accf, stats = _sc_pass_f(srcb, dstb, b3e, tsrc, tdst, zeros_n)
        accb = _sc_pass_b(srcb, dstb, ehat, tdst, zeros_n)
        stats2 = stats.reshape(NW, 128)
        bnh = jnp.stack([lp['bn_h_g'], lp['bn_h_b']])
        bne = jnp.stack([lp['bn_e_g'], lp['bn_e_b']])
        if li < N_LAYERS - 1:
            nxt = gnn[li + 1]
            wn, bn = _pack_node_w(nxt)
            h, ss, tsrc, tdst = _node_upd(
                h, lp['A1_w'].T, lp['A1_b'][None, :], accf, accb, stats2,
                bnh, bne, wn, bn)
            e_cur, b3e = _edge_apply(e_cur, ehat, ss, nxt['B3_w'].T,
                                     nxt['B3_b'][None, :])
        else:
            w1s = p['p1_w'][:, :D_HID].T
            w1d = p['p1_w'][:, D_HID:2 * D_HID].T
            ss, pp = _node_fin(
                h, lp['A1_w'].T, lp['A1_b'][None, :], accf, accb, stats2,
                bnh, bne, x2, w1s, w1d)

    pre = _sc_gather_pre(src, dst, pp)
    w1e = p['p1_w'][:, 2 * D_HID:].T
    scores = _predictor(pre, e_cur, ehat, ss, w1e, p['p1_b'][None, :],
                        p['p2_w'].T, p['p2_b'][None, :])
    return scores


# speedup vs baseline: 1.6540x; 1.6540x over previous
import functools

import jax
import jax.numpy as jnp
from jax import lax
from jax.experimental import pallas as pl
from jax.experimental.pallas import tpu as pltpu
from jax.experimental.pallas import tpu_sc as plsc

N_NODES = 10000
N_EDGES = 320000
D_FEAT = 128
D_EDGE = 16
D_INT = 64
D_HID = 64
N_LAYERS = 4
D_SCORE = 64
L_READ = 64
D_MODEL = 4
D_INNER = 8
D_STATE = 32
D_CONV = 4
DT_RANK = 1

NC = 2
NS = 16
NW = NC * NS
EPT = N_EDGES // NW
CH = 80
NCH = EPT // CH
NPAD = 10240
RPT = NPAD // NS

@functools.cache
def _sc_mesh():
    return plsc.VectorSubcoreMesh(core_axis_name="c", subcore_axis_name="s")


def _sigmoid16(x):
    return 1.0 / (1.0 + jnp.exp(-x))


def _scf_body(src_hbm, dst_hbm, b3e_hbm, tsrc_hbm, tdst_hbm, zero_hbm,
              ehat_hbm, acc_hbm, stats_hbm,
              sidx_v, didx_v, b3e_v, srow_v, drow_v, vals_v, stat_v,
              acc_sh, sem1, sem2, sem3):
    cid = lax.axis_index("c")
    sid = lax.axis_index("s")
    wid = sid * NC + cid
    pltpu.sync_copy(zero_hbm.at[pl.ds(sid * RPT, RPT)],
                    acc_sh.at[pl.ds(sid * RPT, RPT)])
    plsc.subcore_barrier()
    ebase = wid * EPT

    def chunk(c, stats):
        base = ebase + c * CH
        pltpu.sync_copy(src_hbm.at[pl.ds(base, CH)], sidx_v)
        pltpu.sync_copy(dst_hbm.at[pl.ds(base, CH)], didx_v)
        cp1 = pltpu.async_copy(b3e_hbm.at[pl.ds(base, CH)], b3e_v, sem1)
        cp2 = pltpu.async_copy(tsrc_hbm.at[sidx_v], srow_v, sem2)
        cp3 = pltpu.async_copy(tdst_hbm.at[didx_v], drow_v, sem3)
        cp1.wait()
        cp2.wait()
        cp3.wait()

        def row(r, st):
            out = []
            for v in range(4):
                j = v * 16
                b3 = b3e_v[r, pl.ds(j, 16)]
                b1 = srow_v[r, pl.ds(j, 16)]
                a2 = srow_v[r, pl.ds(64 + j, 16)]
                b2 = drow_v[r, pl.ds(j, 16)]
                eh = b3 + b1 + b2
                sg = _sigmoid16(eh)
                b3e_v[r, pl.ds(j, 16)] = eh
                vals_v[r, pl.ds(j, 16)] = sg * a2
                vals_v[r, pl.ds(64 + j, 16)] = sg
                out.append(st[2 * v] + eh)
                out.append(st[2 * v + 1] + eh * eh)
            return tuple(out)

        stats = lax.fori_loop(0, CH, row, stats)
        pltpu.sync_copy(b3e_v, ehat_hbm.at[pl.ds(base, CH)])
        pltpu.sync_copy(vals_v, acc_sh.at[didx_v], add=True)
        return stats

    zero16 = jnp.zeros((16,), jnp.float32)
    stats = lax.fori_loop(0, NCH, chunk, tuple(zero16 for _ in range(8)))
    for v in range(4):
        stat_v[v, :] = stats[2 * v]
        stat_v[4 + v, :] = stats[2 * v + 1]
    pltpu.sync_copy(stat_v, stats_hbm.at[wid])
    plsc.subcore_barrier()
    pltpu.sync_copy(acc_sh.at[pl.ds(sid * RPT, RPT)],
                    acc_hbm.at[pl.ds(cid * NPAD + sid * RPT, RPT)])


def _sc_pass_f(src, dst, b3e, tsrc, tdst, zeros_n):
    fn = pl.kernel(
        _scf_body,
        out_type=[
            jax.ShapeDtypeStruct((N_EDGES, D_HID), jnp.float32),
            jax.ShapeDtypeStruct((NC * NPAD, 128), jnp.float32),
            jax.ShapeDtypeStruct((NW, 8, 16), jnp.float32),
        ],
        mesh=_sc_mesh(),
        scratch_types=[
            pltpu.VMEM((CH,), jnp.int32),
            pltpu.VMEM((CH,), jnp.int32),
            pltpu.VMEM((CH, D_HID), jnp.float32),
            pltpu.VMEM((CH, 128), jnp.float32),
            pltpu.VMEM((CH, 128), jnp.float32),
            pltpu.VMEM((CH, 128), jnp.float32),
            pltpu.VMEM((8, 16), jnp.float32),
            pltpu.VMEM_SHARED((NPAD, 128), jnp.float32),
            pltpu.SemaphoreType.DMA,
            pltpu.SemaphoreType.DMA,
            pltpu.SemaphoreType.DMA,
        ],
    )
    return fn(src, dst, b3e, tsrc, tdst, zeros_n)


def _scb_body(src_hbm, dst_hbm, ehat_hbm, tdst_hbm, zero_hbm,
              acc_hbm,
              sidx_v, didx_v, ehat_v, arow_v,
              acc_sh, sem1, sem2):
    cid = lax.axis_index("c")
    sid = lax.axis_index("s")
    wid = sid * NC + cid
    pltpu.sync_copy(zero_hbm.at[pl.ds(sid * RPT, RPT)],
                    acc_sh.at[pl.ds(sid * RPT, RPT)])
    plsc.subcore_barrier()
    ebase = wid * EPT

    def chunk(c, carry):
        base = ebase + c * CH
        pltpu.sync_copy(src_hbm.at[pl.ds(base, CH)], sidx_v)
        pltpu.sync_copy(dst_hbm.at[pl.ds(base, CH)], didx_v)
        cp1 = pltpu.async_copy(ehat_hbm.at[pl.ds(base, CH)], ehat_v, sem1)
        cp2 = pltpu.async_copy(tdst_hbm.at[didx_v], arow_v, sem2)
        cp1.wait()
        cp2.wait()

        def row(r, cr):
            for v in range(4):
                j = v * 16
                eh = ehat_v[r, pl.ds(j, 16)]
                a3 = arow_v[r, pl.ds(64 + j, 16)]
                sg = _sigmoid16(eh)
                arow_v[r, pl.ds(j, 16)] = sg * a3
                arow_v[r, pl.ds(64 + j, 16)] = sg
            return cr

        lax.fori_loop(0, CH, row, 0)
        pltpu.sync_copy(arow_v, acc_sh.at[sidx_v], add=True)
        return carry

    lax.fori_loop(0, NCH, chunk, 0)
    plsc.subcore_barrier()
    pltpu.sync_copy(acc_sh.at[pl.ds(sid * RPT, RPT)],
                    acc_hbm.at[pl.ds(cid * NPAD + sid * RPT, RPT)])


def _sc_pass_b(src, dst, ehat, tdst, zeros_n):
    fn = pl.kernel(
        _scb_body,
        out_type=[
            jax.ShapeDtypeStruct((NC * NPAD, 128), jnp.float32),
        ],
        mesh=_sc_mesh(),
        scratch_types=[
            pltpu.VMEM((CH,), jnp.int32),
            pltpu.VMEM((CH,), jnp.int32),
            pltpu.VMEM((CH, D_HID), jnp.float32),
            pltpu.VMEM((CH, 128), jnp.float32),
            pltpu.VMEM_SHARED((NPAD, 128), jnp.float32),
            pltpu.SemaphoreType.DMA,
            pltpu.SemaphoreType.DMA,
        ],
    )
    return fn(src, dst, ehat, tdst, zeros_n)[0]


def _scg_body(src_hbm, dst_hbm, pp_hbm,
              pre_hbm,
              sidx_v, didx_v, ps_v, pd_v, out_v, sem1, sem2):
    cid = lax.axis_index("c")
    sid = lax.axis_index("s")
    wid = sid * NC + cid
    ebase = wid * EPT

    def chunk(c, carry):
        base = ebase + c * CH
        pltpu.sync_copy(src_hbm.at[pl.ds(base, CH)], sidx_v)
        pltpu.sync_copy(dst_hbm.at[pl.ds(base, CH)], didx_v)
        cp1 = pltpu.async_copy(pp_hbm.at[sidx_v], ps_v, sem1)
        cp2 = pltpu.async_copy(pp_hbm.at[didx_v], pd_v, sem2)
        cp1.wait()
        cp2.wait()

        def row(r, cr):
            for v in range(4):
                j = v * 16
                out_v[r, pl.ds(j, 16)] = (ps_v[r, pl.ds(j, 16)]
                                          + pd_v[r, pl.ds(64 + j, 16)])
            return cr

        lax.fori_loop(0, CH, row, 0)
        pltpu.sync_copy(out_v, pre_hbm.at[pl.ds(base, CH)])
        return carry

    lax.fori_loop(0, NCH, chunk, 0)


def _sc_gather_pre(src, dst, pp):
    fn = pl.kernel(
        _scg_body,
        out_type=[jax.ShapeDtypeStruct((N_EDGES, D_HID), jnp.float32)],
        mesh=_sc_mesh(),
        scratch_types=[
            pltpu.VMEM((CH,), jnp.int32),
            pltpu.VMEM((CH,), jnp.int32),
            pltpu.VMEM((CH, 128), jnp.float32),
            pltpu.VMEM((CH, 128), jnp.float32),
            pltpu.VMEM((CH, D_HID), jnp.float32),
            pltpu.SemaphoreType.DMA,
            pltpu.SemaphoreType.DMA,
        ],
    )
    return fn(src, dst, pp)[0]


def _node_enc_body(x_ref, w1_ref, b1_ref, w2_ref, b2_ref, wn_ref, bn_ref,
                   h_ref, tsrc_ref, tdst_ref):
    h = jnp.maximum(x_ref[...] @ w1_ref[...] + b1_ref[...], 0.0)
    h = h @ w2_ref[...] + b2_ref[...]
    h_ref[...] = h
    proj = h @ wn_ref[...] + bn_ref[...]
    tsrc_ref[...] = proj[:, :128]
    tdst_ref[...] = proj[:, 128:256]


def _node_enc(x, w1, b1, w2, b2, wn, bn):
    return pl.pallas_call(
        _node_enc_body,
        out_shape=[
            jax.ShapeDtypeStruct((N_NODES, D_HID), jnp.float32),
            jax.ShapeDtypeStruct((N_NODES, 128), jnp.float32),
            jax.ShapeDtypeStruct((N_NODES, 128), jnp.float32),
        ],
    )(x, w1, b1, w2, b2, wn, bn)


_EBLK = 6400
_NEB = N_EDGES // _EBLK


def _edge_enc_body(e_ref, w1_ref, b1_ref, w2_ref, b2_ref, w3_ref, b3_ref,
                   e0_ref, b3e_ref):
    e = jnp.maximum(e_ref[...] @ w1_ref[...] + b1_ref[...], 0.0)
    e = e @ w2_ref[...] + b2_ref[...]
    e0_ref[...] = e
    b3e_ref[...] = e @ w3_ref[...] + b3_ref[...]


def _edge_enc(e, w1, b1, w2, b2, w3, b3):
    blk = lambda i: (i, 0)
    cst = lambda i: (0, 0)
    return pl.pallas_call(
        _edge_enc_body,
        grid=(_NEB,),
        in_specs=[
            pl.BlockSpec((_EBLK, D_EDGE), blk),
            pl.BlockSpec((D_EDGE, D_INT), cst),
            pl.BlockSpec((1, D_INT), cst),
            pl.BlockSpec((D_INT, D_HID), cst),
            pl.BlockSpec((1, D_HID), cst),
            pl.BlockSpec((D_HID, D_HID), cst),
            pl.BlockSpec((1, D_HID), cst),
        ],
        out_specs=[
            pl.BlockSpec((_EBLK, D_HID), blk),
            pl.BlockSpec((_EBLK, D_HID), blk),
        ],
        out_shape=[
            jax.ShapeDtypeStruct((N_EDGES, D_HID), jnp.float32),
            jax.ShapeDtypeStruct((N_EDGES, D_HID), jnp.float32),
        ],
    )(e, w1, b1, w2, b2, w3, b3)


def _node_upd_body(h_ref, a1w_ref, a1b_ref, accf_ref, accb_ref, stats_ref,
                   bnh_ref, bne_ref, wn_ref, bn_ref,
                   h2_ref, ss_ref, tsrc_ref, tdst_ref):
    h = h_ref[...]
    a1h = h @ a1w_ref[...] + a1b_ref[...]
    accf = accf_ref[...]
    accb = accb_ref[...]
    num_f = accf[:N_NODES, :64] + accf[NPAD:NPAD + N_NODES, :64]
    den_f = accf[:N_NODES, 64:] + accf[NPAD:NPAD + N_NODES, 64:]
    num_b = accb[:N_NODES, :64] + accb[NPAD:NPAD + N_NODES, :64]
    den_b = accb[:N_NODES, 64:] + accb[NPAD:NPAD + N_NODES, 64:]
    tmp = a1h + num_f / (den_f + 1e-6) + num_b / (den_b + 1e-6)
    mu = jnp.mean(tmp, axis=0, keepdims=True)
    var = jnp.mean((tmp - mu) ** 2, axis=0, keepdims=True)
    bnh = bnh_ref[...]
    hn = (tmp - mu) / jnp.sqrt(var + 1e-5) * bnh[0:1, :] + bnh[1:2, :]
    h2 = h + jnp.maximum(hn, 0.0)
    h2_ref[...] = h2
    st = jnp.sum(stats_ref[...], axis=0)
    mu_e = st[:64] / N_EDGES
    var_e = st[64:] / N_EDGES - mu_e * mu_e
    bne = bne_ref[...]
    scale = bne[0, :] / jnp.sqrt(var_e + 1e-5)
    shift = bne[1, :] - mu_e * scale
    ss_ref[...] = jnp.concatenate(
        [scale[None, :], shift[None, :], jnp.zeros((6, D_HID), jnp.float32)],
        axis=0)
    proj = h2 @ wn_ref[...] + bn_ref[...]
    tsrc_ref[...] = proj[:, :128]
    tdst_ref[...] = proj[:, 128:256]


def _node_upd(h, a1w, a1b, accf, accb, stats, bnh, bne, wn, bn):
    return pl.pallas_call(
        _node_upd_body,
        out_shape=[
            jax.ShapeDtypeStruct((N_NODES, D_HID), jnp.float32),
            jax.ShapeDtypeStruct((8, D_HID), jnp.float32),
            jax.ShapeDtypeStruct((N_NODES, 128), jnp.float32),
            jax.ShapeDtypeStruct((N_NODES, 128), jnp.float32),
        ],
    )(h, a1w, a1b, accf, accb, stats, bnh, bne, wn, bn)


def _node_fin_body(h_ref, a1w_ref, a1b_ref, accf_ref, accb_ref, stats_ref,
                   bnh_ref, bne_ref, x2_ref, ws_ref, wd_ref,
                   ss_ref, pp_ref):
    h = h_ref[...]
    a1h = h @ a1w_ref[...] + a1b_ref[...]
    accf = accf_ref[...]
    accb = accb_ref[...]
    num_f = accf[:N_NODES, :64] + accf[NPAD:NPAD + N_NODES, :64]
    den_f = accf[:N_NODES, 64:] + accf[NPAD:NPAD + N_NODES, 64:]
    num_b = accb[:N_NODES, :64] + accb[NPAD:NPAD + N_NODES, :64]
    den_b = accb[:N_NODES, 64:] + accb[NPAD:NPAD + N_NODES, 64:]
    tmp = a1h + num_f / (den_f + 1e-6) + num_b / (den_b + 1e-6)
    mu = jnp.mean(tmp, axis=0, keepdims=True)
    var = jnp.mean((tmp - mu) ** 2, axis=0, keepdims=True)
    bnh = bnh_ref[...]
    hn = (tmp - mu) / jnp.sqrt(var + 1e-5) * bnh[0:1, :] + bnh[1:2, :]
    hf = h + jnp.maximum(hn, 0.0) + x2_ref[...]
    st = jnp.sum(stats_ref[...], axis=0)
    mu_e = st[:64] / N_EDGES
    var_e = st[64:] / N_EDGES - mu_e * mu_e
    bne = bne_ref[...]
    scale = bne[0, :] / jnp.sqrt(var_e + 1e-5)
    shift = bne[1, :] - mu_e * scale
    ss_ref[...] = jnp.concatenate(
        [scale[None, :], shift[None, :], jnp.zeros((6, D_HID), jnp.float32)],
        axis=0)
    pp_ref[...] = jnp.concatenate([hf @ ws_ref[...], hf @ wd_ref[...]],
                                  axis=1)


def _node_fin(h, a1w, a1b, accf, accb, stats, bnh, bne, x2, ws, wd):
    return pl.pallas_call(
        _node_fin_body,
        out_shape=[
            jax.ShapeDtypeStruct((8, D_HID), jnp.float32),
            jax.ShapeDtypeStruct((N_NODES, 128), jnp.float32),
        ],
    )(h, a1w, a1b, accf, accb, stats, bnh, bne, x2, ws, wd)


def _edge_apply_body(e_ref, ehat_ref, ss_ref, w_ref, b_ref,
                     e2_ref, b3e_ref):
    ss = ss_ref[...]
    en = e_ref[...] + jnp.maximum(ehat_ref[...] * ss[0:1, :] + ss[1:2, :], 0.0)
    e2_ref[...] = en
    b3e_ref[...] = en @ w_ref[...] + b_ref[...]


def _edge_apply(e, ehat, ss, w, b):
    blk = lambda i: (i, 0)
    cst = lambda i: (0, 0)
    return pl.pallas_call(
        _edge_apply_body,
        grid=(_NEB,),
        in_specs=[
            pl.BlockSpec((_EBLK, D_HID), blk),
            pl.BlockSpec((_EBLK, D_HID), blk),
            pl.BlockSpec((8, D_HID), cst),
            pl.BlockSpec((D_HID, D_HID), cst),
            pl.BlockSpec((1, D_HID), cst),
        ],
        out_specs=[
            pl.BlockSpec((_EBLK, D_HID), blk),
            pl.BlockSpec((_EBLK, D_HID), blk),
        ],
        out_shape=[
            jax.ShapeDtypeStruct((N_EDGES, D_HID), jnp.float32),
            jax.ShapeDtypeStruct((N_EDGES, D_HID), jnp.float32),
        ],
    )(e, ehat, ss, w, b)


def _pred_body(pre_ref, e_ref, ehat_ref, ss_ref, w1e_ref, b1_ref,
               w2_ref, b2_ref, out_ref):
    ss = ss_ref[...]
    e4 = e_ref[...] + jnp.maximum(ehat_ref[...] * ss[0:1, :] + ss[1:2, :], 0.0)
    hcat = pre_ref[...] + e4 @ w1e_ref[...] + b1_ref[...]
    hcat = jnp.maximum(hcat, 0.0)
    out_ref[...] = hcat @ w2_ref[...] + b2_ref[...]


def _predictor(pre, e3, ehat4, ss, w1e, b1, w2, b2):
    blk = lambda i: (i, 0)
    cst = lambda i: (0, 0)
    return pl.pallas_call(
        _pred_body,
        grid=(_NEB,),
        in_specs=[
            pl.BlockSpec((_EBLK, D_HID), blk),
            pl.BlockSpec((_EBLK, D_HID), blk),
            pl.BlockSpec((_EBLK, D_HID), blk),
            pl.BlockSpec((8, D_HID), cst),
            pl.BlockSpec((D_HID, D_SCORE), cst),
            pl.BlockSpec((1, D_SCORE), cst),
            pl.BlockSpec((D_SCORE, 1), cst),
            pl.BlockSpec((1, 1), cst),
        ],
        out_specs=pl.BlockSpec((_EBLK, 1), blk),
        out_shape=jax.ShapeDtypeStruct((N_EDGES, 1), jnp.float32),
    )(pre, e3, ehat4, ss, w1e, b1, w2, b2)


_MBLK = 400
_NMB = N_NODES // _MBLK


def _mamba_body(rd_ref, rl_ref, wx_ref, wz_ref, wc_ref, cb_ref,
                mdt_ref, dtb_ref, wbb_ref, wcb_ref, k8_ref, k8t_ref,
                af_ref, df_ref, esel_ref, fsel_ref,
                wo_ref, wb2_ref, bb2_ref, x2_ref, ys_ref):
    rd = rd_ref[...]
    xm = rd @ wx_ref[...]
    z = rd @ wz_ref[...]
    xc = xm @ wc_ref[...] + cb_ref[...]
    xc = xc * _sigmoid16(xc)
    k8 = k8_ref[...]
    af = af_ref[...]
    h = jnp.zeros((_MBLK, 256), jnp.float32)
    for t in range(L_READ):
        xct = xc[:, t * 8:(t + 1) * 8]
        dpre = xct @ mdt_ref[...] + dtb_ref[...]
        dt = jnp.maximum(dpre, 0.0) + jnp.log1p(jnp.exp(-jnp.abs(dpre)))
        d_bc = dt @ k8
        b_bc = xct @ wbb_ref[...]
        c_bc = xct @ wcb_ref[...]
        u_bc = xct @ k8
        dA = jnp.exp(d_bc * af)
        h = dA * h + d_bc * b_bc * u_bc
        yt = (h * c_bc) @ k8t_ref[...]
        ys_ref[:, t * 8:(t + 1) * 8] = yt
    y = ys_ref[...] + xc * df_ref[...]
    y = y * (z * _sigmoid16(z))
    idx = jnp.clip(rl_ref[0, 0, :] - 1, 0, L_READ - 1)
    tmask = (jax.lax.broadcasted_iota(jnp.int32, (_MBLK, L_READ), 1)
             == idx[:, None]).astype(jnp.float32)
    msel = tmask @ esel_ref[...]
    ylast = (y * msel) @ fsel_ref[...]
    out4 = ylast @ wo_ref[...]
    x2_ref[...] = out4 @ wb2_ref[...] + bb2_ref[...]


def _mamba(rd_flat, rl3, m, base_w, base_b):
    inw = m['in_proj_w']
    wx = jnp.zeros((256, 512), jnp.float32)
    wz = jnp.zeros((256, 512), jnp.float32)
    t_i = jnp.arange(L_READ)
    for mm in range(D_MODEL):
        for d in range(D_INNER):
            wx = wx.at[t_i * 4 + mm, t_i * 8 + d].set(inw[d, mm])
            wz = wz.at[t_i * 4 + mm, t_i * 8 + d].set(inw[D_INNER + d, mm])
    wc = jnp.zeros((512, 512), jnp.float32)
    for k in range(D_CONV):
        tt = jnp.arange(D_CONV - 1 - k, L_READ)
        for d in range(D_INNER):
            wc = wc.at[(tt - (D_CONV - 1 - k)) * 8 + d, tt * 8 + d].set(
                m['conv_w'][d, 0, k])
    cb = jnp.tile(m['conv_b'], (L_READ,))[None, :]
    mdt = m['x_proj_w'][:DT_RANK, :].T @ m['dt_proj_w'].T
    dtb = m['dt_proj_b'][None, :]
    k8 = jnp.zeros((8, 256), jnp.float32)
    d_i = jnp.arange(D_INNER)
    s_i = jnp.arange(D_STATE)
    for s in range(D_STATE):
        k8 = k8.at[d_i, d_i * 32 + s].set(1.0)
    k32 = jnp.zeros((32, 256), jnp.float32)
    for d in range(D_INNER):
        k32 = k32.at[s_i, d * 32 + s_i].set(1.0)
    xpb = m['x_proj_w'][DT_RANK:DT_RANK + D_STATE, :]
    xpc = m['x_proj_w'][DT_RANK + D_STATE:, :]
    wbb = xpb.T @ k32
    wcb = xpc.T @ k32
    af = (-jnp.exp(m['A_log'])).reshape(-1)[None, :]
    df = jnp.tile(m['D'], (L_READ,))[None, :]
    esel = jnp.zeros((L_READ, 512), jnp.float32)
    fsel = jnp.zeros((512, 8), jnp.float32)
    for d in range(D_INNER):
        esel = esel.at[t_i, t_i * 8 + d].set(1.0)
        fsel = fsel.at[t_i * 8 + d, d].set(1.0)
    wo = m['out_proj_w'].T
    wb2 = base_w.T
    bb2 = base_b[None, :]

    blk = lambda i: (i, 0)
    cst = lambda i: (0, 0)
    return pl.pallas_call(
        _mamba_body,
        grid=(_NMB,),
        in_specs=[
            pl.BlockSpec((_MBLK, 256), blk),
            pl.BlockSpec((1, 1, _MBLK), lambda i: (i, 0, 0)),
            pl.BlockSpec((256, 512), cst),
            pl.BlockSpec((256, 512), cst),
            pl.BlockSpec((512, 512), cst),
            pl.BlockSpec((1, 512), cst),
            pl.BlockSpec((8, 8), cst),
            pl.BlockSpec((1, 8), cst),
            pl.BlockSpec((8, 256), cst),
            pl.BlockSpec((8, 256), cst),
            pl.BlockSpec((8, 256), cst),
            pl.BlockSpec((256, 8), cst),
            pl.BlockSpec((1, 256), cst),
            pl.BlockSpec((1, 512), cst),
            pl.BlockSpec((L_READ, 512), cst),
            pl.BlockSpec((512, 8), cst),
            pl.BlockSpec((8, 4), cst),
            pl.BlockSpec((4, D_HID), cst),
            pl.BlockSpec((1, D_HID), cst),
        ],
        out_specs=pl.BlockSpec((_MBLK, D_HID), blk),
        out_shape=jax.ShapeDtypeStruct((N_NODES, D_HID), jnp.float32),
        scratch_shapes=[pltpu.VMEM((_MBLK, 512), jnp.float32)],
    )(rd_flat, rl3, wx, wz, wc, cb, mdt, dtb, wbb, wcb, k8, k8.T,
      af, df, esel, fsel, wo, wb2, bb2)


def _pack_node_w(p):
    wn = jnp.concatenate(
        [p['B1_w'].T, p['A2_w'].T, p['B2_w'].T, p['A3_w'].T], axis=1)
    bn = jnp.concatenate(
        [p['B1_b'], p['A2_b'], p['B2_b'], p['A3_b']])[None, :]
    return wn, bn


def kernel(x, e, edge_index, read_data, read_length, params):
    src = edge_index[0]
    dst = edge_index[1]
    p = params
    gnn = p['gnn']
    zeros_n = jnp.zeros((NPAD, 128), jnp.float32)

    wn1, bn1 = _pack_node_w(gnn[0])
    h, tsrc, tdst = _node_enc(
        x, p['l1n_w'].T, p['l1n_b'][None, :], p['l2n_w'].T, p['l2n_b'][None, :],
        wn1, bn1)
    e_cur, b3e = _edge_enc(
        e, p['l1e_w'].T, p['l1e_b'][None, :], p['l2e_w'].T, p['l2e_b'][None, :],
        gnn[0]['B3_w'].T, gnn[0]['B3_b'][None, :])

    rd_flat = read_data.reshape(N_NODES, L_READ * D_MODEL)
    rl3 = read_length.reshape(_NMB, 1, _MBLK)
    x2 = _mamba(rd_flat, rl3, p['mamba'], p['base_w'], p['base_b'])

    ehat = None
    for li in range(N_LAYERS):
        lp = gnn[li]
        ehat, accf, stats = _sc_pass_f(src, dst, b3e, tsrc, tdst, zeros_n)
        accb = _sc_pass_b(src, dst, ehat, tdst, zeros_n)
        stats2 = stats.reshape(NW, 128)
        bnh = jnp.stack([lp['bn_h_g'], lp['bn_h_b']])
        bne = jnp.stack([lp['bn_e_g'], lp['bn_e_b']])
        if li < N_LAYERS - 1:
            nxt = gnn[li + 1]
            wn, bn = _pack_node_w(nxt)
            h, ss, tsrc, tdst = _node_upd(
                h, lp['A1_w'].T, lp['A1_b'][None, :], accf, accb, stats2,
                bnh, bne, wn, bn)
            e_cur, b3e = _edge_apply(e_cur, ehat, ss, nxt['B3_w'].T,
                                     nxt['B3_b'][None, :])
        else:
            w1s = p['p1_w'][:, :D_HID].T
            w1d = p['p1_w'][:, D_HID:2 * D_HID].T
            ss, pp = _node_fin(
                h, lp['A1_w'].T, lp['A1_b'][None, :], accf, accb, stats2,
                bnh, bne, x2, w1s, w1d)

    pre = _sc_gather_pre(src, dst, pp)
    w1e = p['p1_w'][:, 2 * D_HID:].T
    scores = _predictor(pre, e_cur, ehat, ss, w1e, p['p1_b'][None, :],
                        p['p2_w'].T, p['p2_b'][None, :])
    return scores


# speedup vs baseline: 1.7570x; 1.0623x over previous
import functools

import jax
import jax.numpy as jnp
from jax import lax
from jax.experimental import pallas as pl
from jax.experimental.pallas import tpu as pltpu
from jax.experimental.pallas import tpu_sc as plsc

N_NODES = 10000
N_EDGES = 320000
D_FEAT = 128
D_EDGE = 16
D_INT = 64
D_HID = 64
N_LAYERS = 4
D_SCORE = 64
L_READ = 64
D_MODEL = 4
D_INNER = 8
D_STATE = 32
D_CONV = 4
DT_RANK = 1

NC = 2
NS = 16
NW = NC * NS
EPT = N_EDGES // NW
CH = 80
NCH = EPT // CH
NPAD = 10240
RPT = NPAD // NS

@functools.cache
def _sc_mesh():
    return plsc.VectorSubcoreMesh(core_axis_name="c", subcore_axis_name="s")


def _sigmoid16(x):
    return 1.0 / (1.0 + jnp.exp(-x))


def _scf_body(src_hbm, dst_hbm, b3e_hbm, tsrc_hbm, tdst_hbm, zero_hbm,
              ehat_hbm, acc_hbm, stats_hbm,
              sidx_v, didx_v, b3e_v, srow_v, drow_v, vals_v, stat_v,
              acc_sh, isem1, isem2, sem1, sem2, sem3):
    cid = lax.axis_index("c")
    sid = lax.axis_index("s")
    wid = sid * NC + cid
    pltpu.sync_copy(zero_hbm.at[pl.ds(sid * RPT, RPT)],
                    acc_sh.at[pl.ds(sid * RPT, RPT)])
    plsc.subcore_barrier()
    ebase = wid * EPT

    def issue_gathers(c):
        base = ebase + c * CH
        pltpu.async_copy(b3e_hbm.at[pl.ds(base, CH)], b3e_v, sem1)
        pltpu.async_copy(tsrc_hbm.at[sidx_v], srow_v, sem2)
        pltpu.async_copy(tdst_hbm.at[didx_v], drow_v, sem3)

    def wait_gathers():
        pltpu.make_async_copy(b3e_hbm.at[pl.ds(ebase, CH)], b3e_v, sem1).wait()
        pltpu.make_async_copy(tsrc_hbm.at[sidx_v], srow_v, sem2).wait()
        pltpu.make_async_copy(tdst_hbm.at[didx_v], drow_v, sem3).wait()

    pltpu.sync_copy(src_hbm.at[pl.ds(ebase, CH)], sidx_v)
    pltpu.sync_copy(dst_hbm.at[pl.ds(ebase, CH)], didx_v)
    issue_gathers(0)

    def chunk(c, stats):
        nbase = ebase + jnp.minimum(c + 1, NCH - 1) * CH
        wait_gathers()
        pltpu.async_copy(src_hbm.at[pl.ds(nbase, CH)], sidx_v, isem1)

        def row(r, st):
            out = []
            for v in range(4):
                j = v * 16
                b3 = b3e_v[r, pl.ds(j, 16)]
                b1 = srow_v[r, pl.ds(j, 16)]
                a2 = srow_v[r, pl.ds(64 + j, 16)]
                b2 = drow_v[r, pl.ds(j, 16)]
                eh = b3 + b1 + b2
                sg = _sigmoid16(eh)
                b3e_v[r, pl.ds(j, 16)] = eh
                vals_v[r, pl.ds(j, 16)] = sg * a2
                vals_v[r, pl.ds(64 + j, 16)] = sg
                out.append(st[2 * v] + eh)
                out.append(st[2 * v + 1] + eh * eh)
            return tuple(out)

        stats = lax.fori_loop(0, CH, row, stats)
        base = ebase + c * CH
        pltpu.sync_copy(b3e_v, ehat_hbm.at[pl.ds(base, CH)])
        pltpu.sync_copy(vals_v, acc_sh.at[didx_v], add=True)
        pltpu.async_copy(dst_hbm.at[pl.ds(nbase, CH)], didx_v, isem2)
        pltpu.make_async_copy(src_hbm.at[pl.ds(ebase, CH)], sidx_v, isem1).wait()
        pltpu.make_async_copy(dst_hbm.at[pl.ds(ebase, CH)], didx_v, isem2).wait()
        issue_gathers(jnp.minimum(c + 1, NCH - 1))
        return stats

    zero16 = jnp.zeros((16,), jnp.float32)
    stats = lax.fori_loop(0, NCH, chunk, tuple(zero16 for _ in range(8)))
    wait_gathers()
    for v in range(4):
        stat_v[v, :] = stats[2 * v]
        stat_v[4 + v, :] = stats[2 * v + 1]
    pltpu.sync_copy(stat_v, stats_hbm.at[wid])
    plsc.subcore_barrier()
    pltpu.sync_copy(acc_sh.at[pl.ds(sid * RPT, RPT)],
                    acc_hbm.at[pl.ds(cid * NPAD + sid * RPT, RPT)])


def _sc_pass_f(src, dst, b3e, tsrc, tdst, zeros_n):
    fn = pl.kernel(
        _scf_body,
        out_type=[
            jax.ShapeDtypeStruct((N_EDGES, D_HID), jnp.float32),
            jax.ShapeDtypeStruct((NC * NPAD, 128), jnp.float32),
            jax.ShapeDtypeStruct((NW, 8, 16), jnp.float32),
        ],
        mesh=_sc_mesh(),
        scratch_types=[
            pltpu.VMEM((CH,), jnp.int32),
            pltpu.VMEM((CH,), jnp.int32),
            pltpu.VMEM((CH, D_HID), jnp.float32),
            pltpu.VMEM((CH, 128), jnp.float32),
            pltpu.VMEM((CH, 128), jnp.float32),
            pltpu.VMEM((CH, 128), jnp.float32),
            pltpu.VMEM((8, 16), jnp.float32),
            pltpu.VMEM_SHARED((NPAD, 128), jnp.float32),
            pltpu.SemaphoreType.DMA,
            pltpu.SemaphoreType.DMA,
            pltpu.SemaphoreType.DMA,
            pltpu.SemaphoreType.DMA,
            pltpu.SemaphoreType.DMA,
        ],
    )
    return fn(src, dst, b3e, tsrc, tdst, zeros_n)


def _scb_body(src_hbm, dst_hbm, ehat_hbm, tdst_hbm, zero_hbm,
              acc_hbm,
              sidx_v, didx_v, ehat_v, arow_v,
              acc_sh, isem1, isem2, sem1, sem2):
    cid = lax.axis_index("c")
    sid = lax.axis_index("s")
    wid = sid * NC + cid
    pltpu.sync_copy(zero_hbm.at[pl.ds(sid * RPT, RPT)],
                    acc_sh.at[pl.ds(sid * RPT, RPT)])
    plsc.subcore_barrier()
    ebase = wid * EPT

    def issue_gathers(c):
        base = ebase + c * CH
        pltpu.async_copy(ehat_hbm.at[pl.ds(base, CH)], ehat_v, sem1)
        pltpu.async_copy(tdst_hbm.at[didx_v], arow_v, sem2)

    def wait_gathers():
        pltpu.make_async_copy(ehat_hbm.at[pl.ds(ebase, CH)], ehat_v, sem1).wait()
        pltpu.make_async_copy(tdst_hbm.at[didx_v], arow_v, sem2).wait()

    pltpu.sync_copy(src_hbm.at[pl.ds(ebase, CH)], sidx_v)
    pltpu.sync_copy(dst_hbm.at[pl.ds(ebase, CH)], didx_v)
    issue_gathers(0)

    def chunk(c, carry):
        nbase = ebase + jnp.minimum(c + 1, NCH - 1) * CH
        wait_gathers()
        pltpu.async_copy(dst_hbm.at[pl.ds(nbase, CH)], didx_v, isem2)

        def row(r, cr):
            for v in range(4):
                j = v * 16
                eh = ehat_v[r, pl.ds(j, 16)]
                a3 = arow_v[r, pl.ds(64 + j, 16)]
                sg = _sigmoid16(eh)
                arow_v[r, pl.ds(j, 16)] = sg * a3
                arow_v[r, pl.ds(64 + j, 16)] = sg
            return cr

        lax.fori_loop(0, CH, row, 0)
        pltpu.sync_copy(arow_v, acc_sh.at[sidx_v], add=True)
        pltpu.async_copy(src_hbm.at[pl.ds(nbase, CH)], sidx_v, isem1)
        pltpu.make_async_copy(src_hbm.at[pl.ds(ebase, CH)], sidx_v, isem1).wait()
        pltpu.make_async_copy(dst_hbm.at[pl.ds(ebase, CH)], didx_v, isem2).wait()
        issue_gathers(jnp.minimum(c + 1, NCH - 1))
        return carry

    lax.fori_loop(0, NCH, chunk, 0)
    wait_gathers()
    plsc.subcore_barrier()
    pltpu.sync_copy(acc_sh.at[pl.ds(sid * RPT, RPT)],
                    acc_hbm.at[pl.ds(cid * NPAD + sid * RPT, RPT)])


def _sc_pass_b(src, dst, ehat, tdst, zeros_n):
    fn = pl.kernel(
        _scb_body,
        out_type=[
            jax.ShapeDtypeStruct((NC * NPAD, 128), jnp.float32),
        ],
        mesh=_sc_mesh(),
        scratch_types=[
            pltpu.VMEM((CH,), jnp.int32),
            pltpu.VMEM((CH,), jnp.int32),
            pltpu.VMEM((CH, D_HID), jnp.float32),
            pltpu.VMEM((CH, 128), jnp.float32),
            pltpu.VMEM_SHARED((NPAD, 128), jnp.float32),
            pltpu.SemaphoreType.DMA,
            pltpu.SemaphoreType.DMA,
            pltpu.SemaphoreType.DMA,
            pltpu.SemaphoreType.DMA,
        ],
    )
    return fn(src, dst, ehat, tdst, zeros_n)[0]


def _scg_body(src_hbm, dst_hbm, pp_hbm,
              pre_hbm,
              sidx_v, didx_v, ps_v, pd_v, out_v, sem1, sem2):
    cid = lax.axis_index("c")
    sid = lax.axis_index("s")
    wid = sid * NC + cid
    ebase = wid * EPT

    def chunk(c, carry):
        base = ebase + c * CH
        pltpu.sync_copy(src_hbm.at[pl.ds(base, CH)], sidx_v)
        pltpu.sync_copy(dst_hbm.at[pl.ds(base, CH)], didx_v)
        cp1 = pltpu.async_copy(pp_hbm.at[sidx_v], ps_v, sem1)
        cp2 = pltpu.async_copy(pp_hbm.at[didx_v], pd_v, sem2)
        cp1.wait()
        cp2.wait()

        def row(r, cr):
            for v in range(4):
                j = v * 16
                out_v[r, pl.ds(j, 16)] = (ps_v[r, pl.ds(j, 16)]
                                          + pd_v[r, pl.ds(64 + j, 16)])
            return cr

        lax.fori_loop(0, CH, row, 0)
        pltpu.sync_copy(out_v, pre_hbm.at[pl.ds(base, CH)])
        return carry

    lax.fori_loop(0, NCH, chunk, 0)


def _sc_gather_pre(src, dst, pp):
    fn = pl.kernel(
        _scg_body,
        out_type=[jax.ShapeDtypeStruct((N_EDGES, D_HID), jnp.float32)],
        mesh=_sc_mesh(),
        scratch_types=[
            pltpu.VMEM((CH,), jnp.int32),
            pltpu.VMEM((CH,), jnp.int32),
            pltpu.VMEM((CH, 128), jnp.float32),
            pltpu.VMEM((CH, 128), jnp.float32),
            pltpu.VMEM((CH, D_HID), jnp.float32),
            pltpu.SemaphoreType.DMA,
            pltpu.SemaphoreType.DMA,
        ],
    )
    return fn(src, dst, pp)[0]


def _node_enc_body(x_ref, w1_ref, b1_ref, w2_ref, b2_ref, wn_ref, bn_ref,
                   h_ref, tsrc_ref, tdst_ref):
    h = jnp.maximum(x_ref[...] @ w1_ref[...] + b1_ref[...], 0.0)
    h = h @ w2_ref[...] + b2_ref[...]
    h_ref[...] = h
    proj = h @ wn_ref[...] + bn_ref[...]
    tsrc_ref[...] = proj[:, :128]
    tdst_ref[...] = proj[:, 128:256]


def _node_enc(x, w1, b1, w2, b2, wn, bn):
    return pl.pallas_call(
        _node_enc_body,
        out_shape=[
            jax.ShapeDtypeStruct((N_NODES, D_HID), jnp.float32),
            jax.ShapeDtypeStruct((N_NODES, 128), jnp.float32),
            jax.ShapeDtypeStruct((N_NODES, 128), jnp.float32),
        ],
    )(x, w1, b1, w2, b2, wn, bn)


_EBLK = 6400
_NEB = N_EDGES // _EBLK


def _edge_enc_body(e_ref, w1_ref, b1_ref, w2_ref, b2_ref, w3_ref, b3_ref,
                   e0_ref, b3e_ref):
    e = jnp.maximum(e_ref[...] @ w1_ref[...] + b1_ref[...], 0.0)
    e = e @ w2_ref[...] + b2_ref[...]
    e0_ref[...] = e
    b3e_ref[...] = e @ w3_ref[...] + b3_ref[...]


def _edge_enc(e, w1, b1, w2, b2, w3, b3):
    blk = lambda i: (i, 0)
    cst = lambda i: (0, 0)
    return pl.pallas_call(
        _edge_enc_body,
        grid=(_NEB,),
        in_specs=[
            pl.BlockSpec((_EBLK, D_EDGE), blk),
            pl.BlockSpec((D_EDGE, D_INT), cst),
            pl.BlockSpec((1, D_INT), cst),
            pl.BlockSpec((D_INT, D_HID), cst),
            pl.BlockSpec((1, D_HID), cst),
            pl.BlockSpec((D_HID, D_HID), cst),
            pl.BlockSpec((1, D_HID), cst),
        ],
        out_specs=[
            pl.BlockSpec((_EBLK, D_HID), blk),
            pl.BlockSpec((_EBLK, D_HID), blk),
        ],
        out_shape=[
            jax.ShapeDtypeStruct((N_EDGES, D_HID), jnp.float32),
            jax.ShapeDtypeStruct((N_EDGES, D_HID), jnp.float32),
        ],
    )(e, w1, b1, w2, b2, w3, b3)


def _node_upd_body(h_ref, a1w_ref, a1b_ref, accf_ref, accb_ref, stats_ref,
                   bnh_ref, bne_ref, wn_ref, bn_ref,
                   h2_ref, ss_ref, tsrc_ref, tdst_ref):
    h = h_ref[...]
    a1h = h @ a1w_ref[...] + a1b_ref[...]
    accf = accf_ref[...]
    accb = accb_ref[...]
    num_f = accf[:N_NODES, :64] + accf[NPAD:NPAD + N_NODES, :64]
    den_f = accf[:N_NODES, 64:] + accf[NPAD:NPAD + N_NODES, 64:]
    num_b = accb[:N_NODES, :64] + accb[NPAD:NPAD + N_NODES, :64]
    den_b = accb[:N_NODES, 64:] + accb[NPAD:NPAD + N_NODES, 64:]
    tmp = a1h + num_f / (den_f + 1e-6) + num_b / (den_b + 1e-6)
    mu = jnp.mean(tmp, axis=0, keepdims=True)
    var = jnp.mean((tmp - mu) ** 2, axis=0, keepdims=True)
    bnh = bnh_ref[...]
    hn = (tmp - mu) / jnp.sqrt(var + 1e-5) * bnh[0:1, :] + bnh[1:2, :]
    h2 = h + jnp.maximum(hn, 0.0)
    h2_ref[...] = h2
    st = jnp.sum(stats_ref[...], axis=0)
    mu_e = st[:64] / N_EDGES
    var_e = st[64:] / N_EDGES - mu_e * mu_e
    bne = bne_ref[...]
    scale = bne[0, :] / jnp.sqrt(var_e + 1e-5)
    shift = bne[1, :] - mu_e * scale
    ss_ref[...] = jnp.concatenate(
        [scale[None, :], shift[None, :], jnp.zeros((6, D_HID), jnp.float32)],
        axis=0)
    proj = h2 @ wn_ref[...] + bn_ref[...]
    tsrc_ref[...] = proj[:, :128]
    tdst_ref[...] = proj[:, 128:256]


def _node_upd(h, a1w, a1b, accf, accb, stats, bnh, bne, wn, bn):
    return pl.pallas_call(
        _node_upd_body,
        out_shape=[
            jax.ShapeDtypeStruct((N_NODES, D_HID), jnp.float32),
            jax.ShapeDtypeStruct((8, D_HID), jnp.float32),
            jax.ShapeDtypeStruct((N_NODES, 128), jnp.float32),
            jax.ShapeDtypeStruct((N_NODES, 128), jnp.float32),
        ],
    )(h, a1w, a1b, accf, accb, stats, bnh, bne, wn, bn)


def _node_fin_body(h_ref, a1w_ref, a1b_ref, accf_ref, accb_ref, stats_ref,
                   bnh_ref, bne_ref, x2_ref, ws_ref, wd_ref,
                   ss_ref, pp_ref):
    h = h_ref[...]
    a1h = h @ a1w_ref[...] + a1b_ref[...]
    accf = accf_ref[...]
    accb = accb_ref[...]
    num_f = accf[:N_NODES, :64] + accf[NPAD:NPAD + N_NODES, :64]
    den_f = accf[:N_NODES, 64:] + accf[NPAD:NPAD + N_NODES, 64:]
    num_b = accb[:N_NODES, :64] + accb[NPAD:NPAD + N_NODES, :64]
    den_b = accb[:N_NODES, 64:] + accb[NPAD:NPAD + N_NODES, 64:]
    tmp = a1h + num_f / (den_f + 1e-6) + num_b / (den_b + 1e-6)
    mu = jnp.mean(tmp, axis=0, keepdims=True)
    var = jnp.mean((tmp - mu) ** 2, axis=0, keepdims=True)
    bnh = bnh_ref[...]
    hn = (tmp - mu) / jnp.sqrt(var + 1e-5) * bnh[0:1, :] + bnh[1:2, :]
    hf = h + jnp.maximum(hn, 0.0) + x2_ref[...]
    st = jnp.sum(stats_ref[...], axis=0)
    mu_e = st[:64] / N_EDGES
    var_e = st[64:] / N_EDGES - mu_e * mu_e
    bne = bne_ref[...]
    scale = bne[0, :] / jnp.sqrt(var_e + 1e-5)
    shift = bne[1, :] - mu_e * scale
    ss_ref[...] = jnp.concatenate(
        [scale[None, :], shift[None, :], jnp.zeros((6, D_HID), jnp.float32)],
        axis=0)
    pp_ref[...] = jnp.concatenate([hf @ ws_ref[...], hf @ wd_ref[...]],
                                  axis=1)


def _node_fin(h, a1w, a1b, accf, accb, stats, bnh, bne, x2, ws, wd):
    return pl.pallas_call(
        _node_fin_body,
        out_shape=[
            jax.ShapeDtypeStruct((8, D_HID), jnp.float32),
            jax.ShapeDtypeStruct((N_NODES, 128), jnp.float32),
        ],
    )(h, a1w, a1b, accf, accb, stats, bnh, bne, x2, ws, wd)


def _edge_apply_body(e_ref, ehat_ref, ss_ref, w_ref, b_ref,
                     e2_ref, b3e_ref):
    ss = ss_ref[...]
    en = e_ref[...] + jnp.maximum(ehat_ref[...] * ss[0:1, :] + ss[1:2, :], 0.0)
    e2_ref[...] = en
    b3e_ref[...] = en @ w_ref[...] + b_ref[...]


def _edge_apply(e, ehat, ss, w, b):
    blk = lambda i: (i, 0)
    cst = lambda i: (0, 0)
    return pl.pallas_call(
        _edge_apply_body,
        grid=(_NEB,),
        in_specs=[
            pl.BlockSpec((_EBLK, D_HID), blk),
            pl.BlockSpec((_EBLK, D_HID), blk),
            pl.BlockSpec((8, D_HID), cst),
            pl.BlockSpec((D_HID, D_HID), cst),
            pl.BlockSpec((1, D_HID), cst),
        ],
        out_specs=[
            pl.BlockSpec((_EBLK, D_HID), blk),
            pl.BlockSpec((_EBLK, D_HID), blk),
        ],
        out_shape=[
            jax.ShapeDtypeStruct((N_EDGES, D_HID), jnp.float32),
            jax.ShapeDtypeStruct((N_EDGES, D_HID), jnp.float32),
        ],
    )(e, ehat, ss, w, b)


def _pred_body(pre_ref, e_ref, ehat_ref, ss_ref, w1e_ref, b1_ref,
               w2_ref, b2_ref, out_ref):
    ss = ss_ref[...]
    e4 = e_ref[...] + jnp.maximum(ehat_ref[...] * ss[0:1, :] + ss[1:2, :], 0.0)
    hcat = pre_ref[...] + e4 @ w1e_ref[...] + b1_ref[...]
    hcat = jnp.maximum(hcat, 0.0)
    out_ref[...] = hcat @ w2_ref[...] + b2_ref[...]


def _predictor(pre, e3, ehat4, ss, w1e, b1, w2, b2):
    blk = lambda i: (i, 0)
    cst = lambda i: (0, 0)
    return pl.pallas_call(
        _pred_body,
        grid=(_NEB,),
        in_specs=[
            pl.BlockSpec((_EBLK, D_HID), blk),
            pl.BlockSpec((_EBLK, D_HID), blk),
            pl.BlockSpec((_EBLK, D_HID), blk),
            pl.BlockSpec((8, D_HID), cst),
            pl.BlockSpec((D_HID, D_SCORE), cst),
            pl.BlockSpec((1, D_SCORE), cst),
            pl.BlockSpec((D_SCORE, 1), cst),
            pl.BlockSpec((1, 1), cst),
        ],
        out_specs=pl.BlockSpec((_EBLK, 1), blk),
        out_shape=jax.ShapeDtypeStruct((N_EDGES, 1), jnp.float32),
    )(pre, e3, ehat4, ss, w1e, b1, w2, b2)


_MBLK = 400
_NMB = N_NODES // _MBLK


def _mamba_body(rd_ref, rl_ref, wx_ref, wz_ref, wc_ref, cb_ref,
                mdt_ref, dtb_ref, wbb_ref, wcb_ref, k8_ref, k8t_ref,
                af_ref, df_ref, esel_ref, fsel_ref,
                wo_ref, wb2_ref, bb2_ref, x2_ref, ys_ref):
    rd = rd_ref[...]
    xm = rd @ wx_ref[...]
    z = rd @ wz_ref[...]
    xc = xm @ wc_ref[...] + cb_ref[...]
    xc = xc * _sigmoid16(xc)
    k8 = k8_ref[...]
    af = af_ref[...]
    h = jnp.zeros((_MBLK, 256), jnp.float32)
    for t in range(L_READ):
        xct = xc[:, t * 8:(t + 1) * 8]
        dpre = xct @ mdt_ref[...] + dtb_ref[...]
        dt = jnp.maximum(dpre, 0.0) + jnp.log1p(jnp.exp(-jnp.abs(dpre)))
        d_bc = dt @ k8
        b_bc = xct @ wbb_ref[...]
        c_bc = xct @ wcb_ref[...]
        u_bc = xct @ k8
        dA = jnp.exp(d_bc * af)
        h = dA * h + d_bc * b_bc * u_bc
        yt = (h * c_bc) @ k8t_ref[...]
        ys_ref[:, t * 8:(t + 1) * 8] = yt
    y = ys_ref[...] + xc * df_ref[...]
    y = y * (z * _sigmoid16(z))
    idx = jnp.clip(rl_ref[0, 0, :] - 1, 0, L_READ - 1)
    tmask = (jax.lax.broadcasted_iota(jnp.int32, (_MBLK, L_READ), 1)
             == idx[:, None]).astype(jnp.float32)
    msel = tmask @ esel_ref[...]
    ylast = (y * msel) @ fsel_ref[...]
    out4 = ylast @ wo_ref[...]
    x2_ref[...] = out4 @ wb2_ref[...] + bb2_ref[...]


def _mamba(rd_flat, rl3, m, base_w, base_b):
    inw = m['in_proj_w']
    wx = jnp.zeros((256, 512), jnp.float32)
    wz = jnp.zeros((256, 512), jnp.float32)
    t_i = jnp.arange(L_READ)
    for mm in range(D_MODEL):
        for d in range(D_INNER):
            wx = wx.at[t_i * 4 + mm, t_i * 8 + d].set(inw[d, mm])
            wz = wz.at[t_i * 4 + mm, t_i * 8 + d].set(inw[D_INNER + d, mm])
    wc = jnp.zeros((512, 512), jnp.float32)
    for k in range(D_CONV):
        tt = jnp.arange(D_CONV - 1 - k, L_READ)
        for d in range(D_INNER):
            wc = wc.at[(tt - (D_CONV - 1 - k)) * 8 + d, tt * 8 + d].set(
                m['conv_w'][d, 0, k])
    cb = jnp.tile(m['conv_b'], (L_READ,))[None, :]
    mdt = m['x_proj_w'][:DT_RANK, :].T @ m['dt_proj_w'].T
    dtb = m['dt_proj_b'][None, :]
    k8 = jnp.zeros((8, 256), jnp.float32)
    d_i = jnp.arange(D_INNER)
    s_i = jnp.arange(D_STATE)
    for s in range(D_STATE):
        k8 = k8.at[d_i, d_i * 32 + s].set(1.0)
    k32 = jnp.zeros((32, 256), jnp.float32)
    for d in range(D_INNER):
        k32 = k32.at[s_i, d * 32 + s_i].set(1.0)
    xpb = m['x_proj_w'][DT_RANK:DT_RANK + D_STATE, :]
    xpc = m['x_proj_w'][DT_RANK + D_STATE:, :]
    wbb = xpb.T @ k32
    wcb = xpc.T @ k32
    af = (-jnp.exp(m['A_log'])).reshape(-1)[None, :]
    df = jnp.tile(m['D'], (L_READ,))[None, :]
    esel = jnp.zeros((L_READ, 512), jnp.float32)
    fsel = jnp.zeros((512, 8), jnp.float32)
    for d in range(D_INNER):
        esel = esel.at[t_i, t_i * 8 + d].set(1.0)
        fsel = fsel.at[t_i * 8 + d, d].set(1.0)
    wo = m['out_proj_w'].T
    wb2 = base_w.T
    bb2 = base_b[None, :]

    blk = lambda i: (i, 0)
    cst = lambda i: (0, 0)
    return pl.pallas_call(
        _mamba_body,
        grid=(_NMB,),
        in_specs=[
            pl.BlockSpec((_MBLK, 256), blk),
            pl.BlockSpec((1, 1, _MBLK), lambda i: (i, 0, 0)),
            pl.BlockSpec((256, 512), cst),
            pl.BlockSpec((256, 512), cst),
            pl.BlockSpec((512, 512), cst),
            pl.BlockSpec((1, 512), cst),
            pl.BlockSpec((8, 8), cst),
            pl.BlockSpec((1, 8), cst),
            pl.BlockSpec((8, 256), cst),
            pl.BlockSpec((8, 256), cst),
            pl.BlockSpec((8, 256), cst),
            pl.BlockSpec((256, 8), cst),
            pl.BlockSpec((1, 256), cst),
            pl.BlockSpec((1, 512), cst),
            pl.BlockSpec((L_READ, 512), cst),
            pl.BlockSpec((512, 8), cst),
            pl.BlockSpec((8, 4), cst),
            pl.BlockSpec((4, D_HID), cst),
            pl.BlockSpec((1, D_HID), cst),
        ],
        out_specs=pl.BlockSpec((_MBLK, D_HID), blk),
        out_shape=jax.ShapeDtypeStruct((N_NODES, D_HID), jnp.float32),
        scratch_shapes=[pltpu.VMEM((_MBLK, 512), jnp.float32)],
    )(rd_flat, rl3, wx, wz, wc, cb, mdt, dtb, wbb, wcb, k8, k8.T,
      af, df, esel, fsel, wo, wb2, bb2)


def _pack_node_w(p):
    wn = jnp.concatenate(
        [p['B1_w'].T, p['A2_w'].T, p['B2_w'].T, p['A3_w'].T], axis=1)
    bn = jnp.concatenate(
        [p['B1_b'], p['A2_b'], p['B2_b'], p['A3_b']])[None, :]
    return wn, bn


def kernel(x, e, edge_index, read_data, read_length, params):
    src = edge_index[0]
    dst = edge_index[1]
    p = params
    gnn = p['gnn']
    zeros_n = jnp.zeros((NPAD, 128), jnp.float32)

    wn1, bn1 = _pack_node_w(gnn[0])
    h, tsrc, tdst = _node_enc(
        x, p['l1n_w'].T, p['l1n_b'][None, :], p['l2n_w'].T, p['l2n_b'][None, :],
        wn1, bn1)
    e_cur, b3e = _edge_enc(
        e, p['l1e_w'].T, p['l1e_b'][None, :], p['l2e_w'].T, p['l2e_b'][None, :],
        gnn[0]['B3_w'].T, gnn[0]['B3_b'][None, :])

    rd_flat = read_data.reshape(N_NODES, L_READ * D_MODEL)
    rl3 = read_length.reshape(_NMB, 1, _MBLK)
    x2 = _mamba(rd_flat, rl3, p['mamba'], p['base_w'], p['base_b'])

    ehat = None
    for li in range(N_LAYERS):
        lp = gnn[li]
        ehat, accf, stats = _sc_pass_f(src, dst, b3e, tsrc, tdst, zeros_n)
        accb = _sc_pass_b(src, dst, ehat, tdst, zeros_n)
        stats2 = stats.reshape(NW, 128)
        bnh = jnp.stack([lp['bn_h_g'], lp['bn_h_b']])
        bne = jnp.stack([lp['bn_e_g'], lp['bn_e_b']])
        if li < N_LAYERS - 1:
            nxt = gnn[li + 1]
            wn, bn = _pack_node_w(nxt)
            h, ss, tsrc, tdst = _node_upd(
                h, lp['A1_w'].T, lp['A1_b'][None, :], accf, accb, stats2,
                bnh, bne, wn, bn)
            e_cur, b3e = _edge_apply(e_cur, ehat, ss, nxt['B3_w'].T,
                                     nxt['B3_b'][None, :])
        else:
            w1s = p['p1_w'][:, :D_HID].T
            w1d = p['p1_w'][:, D_HID:2 * D_HID].T
            ss, pp = _node_fin(
                h, lp['A1_w'].T, lp['A1_b'][None, :], accf, accb, stats2,
                bnh, bne, x2, w1s, w1d)

    pre = _sc_gather_pre(src, dst, pp)
    w1e = p['p1_w'][:, 2 * D_HID:].T
    scores = _predictor(pre, e_cur, ehat, ss, w1e, p['p1_b'][None, :],
                        p['p2_w'].T, p['p2_b'][None, :])
    return scores


# speedup vs baseline: 1.8040x; 1.0268x over previous
import functools

import jax
import jax.numpy as jnp
from jax import lax
from jax.experimental import pallas as pl
from jax.experimental.pallas import tpu as pltpu
from jax.experimental.pallas import tpu_sc as plsc

N_NODES = 10000
N_EDGES = 320000
D_FEAT = 128
D_EDGE = 16
D_INT = 64
D_HID = 64
N_LAYERS = 4
D_SCORE = 64
L_READ = 64
D_MODEL = 4
D_INNER = 8
D_STATE = 32
D_CONV = 4
DT_RANK = 1

NC = 2
NS = 16
NW = NC * NS
EPT = N_EDGES // NW
CH = 80
NCH = EPT // CH
NPAD = 10240
RPT = NPAD // NS

@functools.cache
def _sc_mesh():
    return plsc.VectorSubcoreMesh(core_axis_name="c", subcore_axis_name="s")


def _sigmoid16(x):
    return 1.0 / (1.0 + jnp.exp(-x))


def _scf_body(src_hbm, dst_hbm, b3e_hbm, tsrc_hbm, tdst_hbm, zero_hbm,
              ehat_hbm, acc_hbm, stats_hbm,
              sidx_v, didx_v, b3e_v, srow_v, drow_v, vals_v, stat_v,
              acc_sh, isem1, isem2, sem1, sem2, sem3, wsem):
    cid = lax.axis_index("c")
    sid = lax.axis_index("s")
    wid = sid * NC + cid
    pltpu.sync_copy(zero_hbm.at[pl.ds(sid * RPT, RPT)],
                    acc_sh.at[pl.ds(sid * RPT, RPT)])
    plsc.subcore_barrier()
    ebase = wid * EPT

    def issue_gathers(c):
        base = ebase + c * CH
        pltpu.async_copy(b3e_hbm.at[pl.ds(base, CH)], b3e_v, sem1)
        pltpu.async_copy(tsrc_hbm.at[sidx_v], srow_v, sem2)
        pltpu.async_copy(tdst_hbm.at[didx_v], drow_v, sem3)

    def wait_gathers():
        pltpu.make_async_copy(b3e_hbm.at[pl.ds(ebase, CH)], b3e_v, sem1).wait()
        pltpu.make_async_copy(tsrc_hbm.at[sidx_v], srow_v, sem2).wait()
        pltpu.make_async_copy(tdst_hbm.at[didx_v], drow_v, sem3).wait()

    pltpu.sync_copy(src_hbm.at[pl.ds(ebase, CH)], sidx_v)
    pltpu.sync_copy(dst_hbm.at[pl.ds(ebase, CH)], didx_v)
    issue_gathers(0)

    def chunk(c, stats):
        nbase = ebase + jnp.minimum(c + 1, NCH - 1) * CH
        wait_gathers()
        pltpu.async_copy(src_hbm.at[pl.ds(nbase, CH)], sidx_v, isem1)

        def row(r, st):
            out = []
            for v in range(4):
                j = v * 16
                b3 = b3e_v[r, pl.ds(j, 16)]
                b1 = srow_v[r, pl.ds(j, 16)]
                a2 = srow_v[r, pl.ds(64 + j, 16)]
                b2 = drow_v[r, pl.ds(j, 16)]
                eh = b3 + b1 + b2
                sg = _sigmoid16(eh)
                b3e_v[r, pl.ds(j, 16)] = eh
                vals_v[r, pl.ds(j, 16)] = sg * a2
                vals_v[r, pl.ds(64 + j, 16)] = sg
                out.append(st[2 * v] + eh)
                out.append(st[2 * v + 1] + eh * eh)
            return tuple(out)

        stats = lax.fori_loop(0, CH, row, stats)
        base = ebase + c * CH
        pltpu.async_copy(b3e_v, ehat_hbm.at[pl.ds(base, CH)], wsem)
        pltpu.sync_copy(vals_v, acc_sh.at[didx_v], add=True)
        pltpu.async_copy(dst_hbm.at[pl.ds(nbase, CH)], didx_v, isem2)
        pltpu.make_async_copy(src_hbm.at[pl.ds(ebase, CH)], sidx_v, isem1).wait()
        pltpu.make_async_copy(dst_hbm.at[pl.ds(ebase, CH)], didx_v, isem2).wait()
        pltpu.make_async_copy(b3e_v, ehat_hbm.at[pl.ds(ebase, CH)], wsem).wait()
        issue_gathers(jnp.minimum(c + 1, NCH - 1))
        return stats

    zero16 = jnp.zeros((16,), jnp.float32)
    stats = lax.fori_loop(0, NCH, chunk, tuple(zero16 for _ in range(8)))
    wait_gathers()
    for v in range(4):
        stat_v[v, :] = stats[2 * v]
        stat_v[4 + v, :] = stats[2 * v + 1]
    pltpu.sync_copy(stat_v, stats_hbm.at[wid])
    plsc.subcore_barrier()
    pltpu.sync_copy(acc_sh.at[pl.ds(sid * RPT, RPT)],
                    acc_hbm.at[pl.ds(cid * NPAD + sid * RPT, RPT)])


def _sc_pass_f(src, dst, b3e, tsrc, tdst, zeros_n):
    fn = pl.kernel(
        _scf_body,
        out_type=[
            jax.ShapeDtypeStruct((N_EDGES, D_HID), jnp.float32),
            jax.ShapeDtypeStruct((NC * NPAD, 128), jnp.float32),
            jax.ShapeDtypeStruct((NW, 8, 16), jnp.float32),
        ],
        mesh=_sc_mesh(),
        scratch_types=[
            pltpu.VMEM((CH,), jnp.int32),
            pltpu.VMEM((CH,), jnp.int32),
            pltpu.VMEM((CH, D_HID), jnp.float32),
            pltpu.VMEM((CH, 128), jnp.float32),
            pltpu.VMEM((CH, 128), jnp.float32),
            pltpu.VMEM((CH, 128), jnp.float32),
            pltpu.VMEM((8, 16), jnp.float32),
            pltpu.VMEM_SHARED((NPAD, 128), jnp.float32),
            pltpu.SemaphoreType.DMA,
            pltpu.SemaphoreType.DMA,
            pltpu.SemaphoreType.DMA,
            pltpu.SemaphoreType.DMA,
            pltpu.SemaphoreType.DMA,
            pltpu.SemaphoreType.DMA,
        ],
    )
    return fn(src, dst, b3e, tsrc, tdst, zeros_n)


def _scb_body(src_hbm, dst_hbm, ehat_hbm, tdst_hbm, zero_hbm,
              acc_hbm,
              sidx_v, didx_v, ehat_v, arow_v,
              acc_sh, isem1, isem2, sem1, sem2):
    cid = lax.axis_index("c")
    sid = lax.axis_index("s")
    wid = sid * NC + cid
    pltpu.sync_copy(zero_hbm.at[pl.ds(sid * RPT, RPT)],
                    acc_sh.at[pl.ds(sid * RPT, RPT)])
    plsc.subcore_barrier()
    ebase = wid * EPT

    def issue_gathers(c):
        base = ebase + c * CH
        pltpu.async_copy(ehat_hbm.at[pl.ds(base, CH)], ehat_v, sem1)
        pltpu.async_copy(tdst_hbm.at[didx_v], arow_v, sem2)

    def wait_gathers():
        pltpu.make_async_copy(ehat_hbm.at[pl.ds(ebase, CH)], ehat_v, sem1).wait()
        pltpu.make_async_copy(tdst_hbm.at[didx_v], arow_v, sem2).wait()

    pltpu.sync_copy(src_hbm.at[pl.ds(ebase, CH)], sidx_v)
    pltpu.sync_copy(dst_hbm.at[pl.ds(ebase, CH)], didx_v)
    issue_gathers(0)

    def chunk(c, carry):
        nbase = ebase + jnp.minimum(c + 1, NCH - 1) * CH
        wait_gathers()
        pltpu.async_copy(dst_hbm.at[pl.ds(nbase, CH)], didx_v, isem2)

        def row(r, cr):
            for v in range(4):
                j = v * 16
                eh = ehat_v[r, pl.ds(j, 16)]
                a3 = arow_v[r, pl.ds(64 + j, 16)]
                sg = _sigmoid16(eh)
                arow_v[r, pl.ds(j, 16)] = sg * a3
                arow_v[r, pl.ds(64 + j, 16)] = sg
            return cr

        lax.fori_loop(0, CH, row, 0)
        pltpu.sync_copy(arow_v, acc_sh.at[sidx_v], add=True)
        pltpu.async_copy(src_hbm.at[pl.ds(nbase, CH)], sidx_v, isem1)
        pltpu.make_async_copy(src_hbm.at[pl.ds(ebase, CH)], sidx_v, isem1).wait()
        pltpu.make_async_copy(dst_hbm.at[pl.ds(ebase, CH)], didx_v, isem2).wait()
        issue_gathers(jnp.minimum(c + 1, NCH - 1))
        return carry

    lax.fori_loop(0, NCH, chunk, 0)
    wait_gathers()
    plsc.subcore_barrier()
    pltpu.sync_copy(acc_sh.at[pl.ds(sid * RPT, RPT)],
                    acc_hbm.at[pl.ds(cid * NPAD + sid * RPT, RPT)])


def _sc_pass_b(src, dst, ehat, tdst, zeros_n):
    fn = pl.kernel(
        _scb_body,
        out_type=[
            jax.ShapeDtypeStruct((NC * NPAD, 128), jnp.float32),
        ],
        mesh=_sc_mesh(),
        scratch_types=[
            pltpu.VMEM((CH,), jnp.int32),
            pltpu.VMEM((CH,), jnp.int32),
            pltpu.VMEM((CH, D_HID), jnp.float32),
            pltpu.VMEM((CH, 128), jnp.float32),
            pltpu.VMEM_SHARED((NPAD, 128), jnp.float32),
            pltpu.SemaphoreType.DMA,
            pltpu.SemaphoreType.DMA,
            pltpu.SemaphoreType.DMA,
            pltpu.SemaphoreType.DMA,
        ],
    )
    return fn(src, dst, ehat, tdst, zeros_n)[0]


def _scg_body(src_hbm, dst_hbm, pp_hbm,
              pre_hbm,
              sidx_v, didx_v, ps_v, pd_v, out_v, sem1, sem2):
    cid = lax.axis_index("c")
    sid = lax.axis_index("s")
    wid = sid * NC + cid
    ebase = wid * EPT

    def chunk(c, carry):
        base = ebase + c * CH
        pltpu.sync_copy(src_hbm.at[pl.ds(base, CH)], sidx_v)
        pltpu.sync_copy(dst_hbm.at[pl.ds(base, CH)], didx_v)
        cp1 = pltpu.async_copy(pp_hbm.at[sidx_v], ps_v, sem1)
        cp2 = pltpu.async_copy(pp_hbm.at[didx_v], pd_v, sem2)
        cp1.wait()
        cp2.wait()

        def row(r, cr):
            for v in range(4):
                j = v * 16
                out_v[r, pl.ds(j, 16)] = (ps_v[r, pl.ds(j, 16)]
                                          + pd_v[r, pl.ds(64 + j, 16)])
            return cr

        lax.fori_loop(0, CH, row, 0)
        pltpu.sync_copy(out_v, pre_hbm.at[pl.ds(base, CH)])
        return carry

    lax.fori_loop(0, NCH, chunk, 0)


def _sc_gather_pre(src, dst, pp):
    fn = pl.kernel(
        _scg_body,
        out_type=[jax.ShapeDtypeStruct((N_EDGES, D_HID), jnp.float32)],
        mesh=_sc_mesh(),
        scratch_types=[
            pltpu.VMEM((CH,), jnp.int32),
            pltpu.VMEM((CH,), jnp.int32),
            pltpu.VMEM((CH, 128), jnp.float32),
            pltpu.VMEM((CH, 128), jnp.float32),
            pltpu.VMEM((CH, D_HID), jnp.float32),
            pltpu.SemaphoreType.DMA,
            pltpu.SemaphoreType.DMA,
        ],
    )
    return fn(src, dst, pp)[0]


def _node_enc_body(x_ref, w1_ref, b1_ref, w2_ref, b2_ref, wn_ref, bn_ref,
                   h_ref, tsrc_ref, tdst_ref):
    h = jnp.maximum(x_ref[...] @ w1_ref[...] + b1_ref[...], 0.0)
    h = h @ w2_ref[...] + b2_ref[...]
    h_ref[...] = h
    proj = h @ wn_ref[...] + bn_ref[...]
    tsrc_ref[...] = proj[:, :128]
    tdst_ref[...] = proj[:, 128:256]


def _node_enc(x, w1, b1, w2, b2, wn, bn):
    return pl.pallas_call(
        _node_enc_body,
        out_shape=[
            jax.ShapeDtypeStruct((N_NODES, D_HID), jnp.float32),
            jax.ShapeDtypeStruct((N_NODES, 128), jnp.float32),
            jax.ShapeDtypeStruct((N_NODES, 128), jnp.float32),
        ],
    )(x, w1, b1, w2, b2, wn, bn)


_EBLK = 6400
_NEB = N_EDGES // _EBLK


def _edge_enc_body(e_ref, w1_ref, b1_ref, w2_ref, b2_ref, w3_ref, b3_ref,
                   e0_ref, b3e_ref):
    e = jnp.maximum(e_ref[...] @ w1_ref[...] + b1_ref[...], 0.0)
    e = e @ w2_ref[...] + b2_ref[...]
    e0_ref[...] = e
    b3e_ref[...] = e @ w3_ref[...] + b3_ref[...]


def _edge_enc(e, w1, b1, w2, b2, w3, b3):
    blk = lambda i: (i, 0)
    cst = lambda i: (0, 0)
    return pl.pallas_call(
        _edge_enc_body,
        grid=(_NEB,),
        in_specs=[
            pl.BlockSpec((_EBLK, D_EDGE), blk),
            pl.BlockSpec((D_EDGE, D_INT), cst),
            pl.BlockSpec((1, D_INT), cst),
            pl.BlockSpec((D_INT, D_HID), cst),
            pl.BlockSpec((1, D_HID), cst),
            pl.BlockSpec((D_HID, D_HID), cst),
            pl.BlockSpec((1, D_HID), cst),
        ],
        out_specs=[
            pl.BlockSpec((_EBLK, D_HID), blk),
            pl.BlockSpec((_EBLK, D_HID), blk),
        ],
        out_shape=[
            jax.ShapeDtypeStruct((N_EDGES, D_HID), jnp.float32),
            jax.ShapeDtypeStruct((N_EDGES, D_HID), jnp.float32),
        ],
    )(e, w1, b1, w2, b2, w3, b3)


def _node_upd_body(h_ref, a1w_ref, a1b_ref, accf_ref, accb_ref, stats_ref,
                   bnh_ref, bne_ref, wn_ref, bn_ref,
                   h2_ref, ss_ref, tsrc_ref, tdst_ref):
    h = h_ref[...]
    a1h = h @ a1w_ref[...] + a1b_ref[...]
    accf = accf_ref[...]
    accb = accb_ref[...]
    num_f = accf[:N_NODES, :64] + accf[NPAD:NPAD + N_NODES, :64]
    den_f = accf[:N_NODES, 64:] + accf[NPAD:NPAD + N_NODES, 64:]
    num_b = accb[:N_NODES, :64] + accb[NPAD:NPAD + N_NODES, :64]
    den_b = accb[:N_NODES, 64:] + accb[NPAD:NPAD + N_NODES, 64:]
    tmp = a1h + num_f / (den_f + 1e-6) + num_b / (den_b + 1e-6)
    mu = jnp.mean(tmp, axis=0, keepdims=True)
    var = jnp.mean((tmp - mu) ** 2, axis=0, keepdims=True)
    bnh = bnh_ref[...]
    hn = (tmp - mu) / jnp.sqrt(var + 1e-5) * bnh[0:1, :] + bnh[1:2, :]
    h2 = h + jnp.maximum(hn, 0.0)
    h2_ref[...] = h2
    st = jnp.sum(stats_ref[...], axis=0)
    mu_e = st[:64] / N_EDGES
    var_e = st[64:] / N_EDGES - mu_e * mu_e
    bne = bne_ref[...]
    scale = bne[0, :] / jnp.sqrt(var_e + 1e-5)
    shift = bne[1, :] - mu_e * scale
    ss_ref[...] = jnp.concatenate(
        [scale[None, :], shift[None, :], jnp.zeros((6, D_HID), jnp.float32)],
        axis=0)
    proj = h2 @ wn_ref[...] + bn_ref[...]
    tsrc_ref[...] = proj[:, :128]
    tdst_ref[...] = proj[:, 128:256]


def _node_upd(h, a1w, a1b, accf, accb, stats, bnh, bne, wn, bn):
    return pl.pallas_call(
        _node_upd_body,
        out_shape=[
            jax.ShapeDtypeStruct((N_NODES, D_HID), jnp.float32),
            jax.ShapeDtypeStruct((8, D_HID), jnp.float32),
            jax.ShapeDtypeStruct((N_NODES, 128), jnp.float32),
            jax.ShapeDtypeStruct((N_NODES, 128), jnp.float32),
        ],
    )(h, a1w, a1b, accf, accb, stats, bnh, bne, wn, bn)


def _node_fin_body(h_ref, a1w_ref, a1b_ref, accf_ref, accb_ref, stats_ref,
                   bnh_ref, bne_ref, x2_ref, ws_ref, wd_ref,
                   ss_ref, pp_ref):
    h = h_ref[...]
    a1h = h @ a1w_ref[...] + a1b_ref[...]
    accf = accf_ref[...]
    accb = accb_ref[...]
    num_f = accf[:N_NODES, :64] + accf[NPAD:NPAD + N_NODES, :64]
    den_f = accf[:N_NODES, 64:] + accf[NPAD:NPAD + N_NODES, 64:]
    num_b = accb[:N_NODES, :64] + accb[NPAD:NPAD + N_NODES, :64]
    den_b = accb[:N_NODES, 64:] + accb[NPAD:NPAD + N_NODES, 64:]
    tmp = a1h + num_f / (den_f + 1e-6) + num_b / (den_b + 1e-6)
    mu = jnp.mean(tmp, axis=0, keepdims=True)
    var = jnp.mean((tmp - mu) ** 2, axis=0, keepdims=True)
    bnh = bnh_ref[...]
    hn = (tmp - mu) / jnp.sqrt(var + 1e-5) * bnh[0:1, :] + bnh[1:2, :]
    hf = h + jnp.maximum(hn, 0.0) + x2_ref[...]
    st = jnp.sum(stats_ref[...], axis=0)
    mu_e = st[:64] / N_EDGES
    var_e = st[64:] / N_EDGES - mu_e * mu_e
    bne = bne_ref[...]
    scale = bne[0, :] / jnp.sqrt(var_e + 1e-5)
    shift = bne[1, :] - mu_e * scale
    ss_ref[...] = jnp.concatenate(
        [scale[None, :], shift[None, :], jnp.zeros((6, D_HID), jnp.float32)],
        axis=0)
    pp_ref[...] = jnp.concatenate([hf @ ws_ref[...], hf @ wd_ref[...]],
                                  axis=1)


def _node_fin(h, a1w, a1b, accf, accb, stats, bnh, bne, x2, ws, wd):
    return pl.pallas_call(
        _node_fin_body,
        out_shape=[
            jax.ShapeDtypeStruct((8, D_HID), jnp.float32),
            jax.ShapeDtypeStruct((N_NODES, 128), jnp.float32),
        ],
    )(h, a1w, a1b, accf, accb, stats, bnh, bne, x2, ws, wd)


def _edge_apply_body(e_ref, ehat_ref, ss_ref, w_ref, b_ref,
                     e2_ref, b3e_ref):
    ss = ss_ref[...]
    en = e_ref[...] + jnp.maximum(ehat_ref[...] * ss[0:1, :] + ss[1:2, :], 0.0)
    e2_ref[...] = en
    b3e_ref[...] = en @ w_ref[...] + b_ref[...]


def _edge_apply(e, ehat, ss, w, b):
    blk = lambda i: (i, 0)
    cst = lambda i: (0, 0)
    return pl.pallas_call(
        _edge_apply_body,
        grid=(_NEB,),
        in_specs=[
            pl.BlockSpec((_EBLK, D_HID), blk),
            pl.BlockSpec((_EBLK, D_HID), blk),
            pl.BlockSpec((8, D_HID), cst),
            pl.BlockSpec((D_HID, D_HID), cst),
            pl.BlockSpec((1, D_HID), cst),
        ],
        out_specs=[
            pl.BlockSpec((_EBLK, D_HID), blk),
            pl.BlockSpec((_EBLK, D_HID), blk),
        ],
        out_shape=[
            jax.ShapeDtypeStruct((N_EDGES, D_HID), jnp.float32),
            jax.ShapeDtypeStruct((N_EDGES, D_HID), jnp.float32),
        ],
    )(e, ehat, ss, w, b)


def _pred_body(pre_ref, e_ref, ehat_ref, ss_ref, w1e_ref, b1_ref,
               w2_ref, b2_ref, out_ref):
    ss = ss_ref[...]
    e4 = e_ref[...] + jnp.maximum(ehat_ref[...] * ss[0:1, :] + ss[1:2, :], 0.0)
    hcat = pre_ref[...] + e4 @ w1e_ref[...] + b1_ref[...]
    hcat = jnp.maximum(hcat, 0.0)
    out_ref[...] = hcat @ w2_ref[...] + b2_ref[...]


def _predictor(pre, e3, ehat4, ss, w1e, b1, w2, b2):
    blk = lambda i: (i, 0)
    cst = lambda i: (0, 0)
    return pl.pallas_call(
        _pred_body,
        grid=(_NEB,),
        in_specs=[
            pl.BlockSpec((_EBLK, D_HID), blk),
            pl.BlockSpec((_EBLK, D_HID), blk),
            pl.BlockSpec((_EBLK, D_HID), blk),
            pl.BlockSpec((8, D_HID), cst),
            pl.BlockSpec((D_HID, D_SCORE), cst),
            pl.BlockSpec((1, D_SCORE), cst),
            pl.BlockSpec((D_SCORE, 1), cst),
            pl.BlockSpec((1, 1), cst),
        ],
        out_specs=pl.BlockSpec((_EBLK, 1), blk),
        out_shape=jax.ShapeDtypeStruct((N_EDGES, 1), jnp.float32),
    )(pre, e3, ehat4, ss, w1e, b1, w2, b2)


_MBLK = 400
_NMB = N_NODES // _MBLK


def _mamba_body(rd_ref, rl_ref, wx_ref, wz_ref, wc_ref, cb_ref,
                mdt_ref, dtb_ref, wbb_ref, wcb_ref, k8_ref, k8t_ref,
                af_ref, df_ref, esel_ref, fsel_ref,
                wo_ref, wb2_ref, bb2_ref, x2_ref, ys_ref):
    rd = rd_ref[...]
    xm = rd @ wx_ref[...]
    z = rd @ wz_ref[...]
    xc = xm @ wc_ref[...] + cb_ref[...]
    xc = xc * _sigmoid16(xc)
    k8 = k8_ref[...]
    af = af_ref[...]
    h = jnp.zeros((_MBLK, 256), jnp.float32)
    for t in range(L_READ):
        xct = xc[:, t * 8:(t + 1) * 8]
        dpre = xct @ mdt_ref[...] + dtb_ref[...]
        dt = jnp.maximum(dpre, 0.0) + jnp.log1p(jnp.exp(-jnp.abs(dpre)))
        d_bc = dt @ k8
        b_bc = xct @ wbb_ref[...]
        c_bc = xct @ wcb_ref[...]
        u_bc = xct @ k8
        dA = jnp.exp(d_bc * af)
        h = dA * h + d_bc * b_bc * u_bc
        yt = (h * c_bc) @ k8t_ref[...]
        ys_ref[:, t * 8:(t + 1) * 8] = yt
    y = ys_ref[...] + xc * df_ref[...]
    y = y * (z * _sigmoid16(z))
    idx = jnp.clip(rl_ref[0, 0, :] - 1, 0, L_READ - 1)
    tmask = (jax.lax.broadcasted_iota(jnp.int32, (_MBLK, L_READ), 1)
             == idx[:, None]).astype(jnp.float32)
    msel = tmask @ esel_ref[...]
    ylast = (y * msel) @ fsel_ref[...]
    out4 = ylast @ wo_ref[...]
    x2_ref[...] = out4 @ wb2_ref[...] + bb2_ref[...]


def _mamba(rd_flat, rl3, m, base_w, base_b):
    inw = m['in_proj_w']
    wx = jnp.zeros((256, 512), jnp.float32)
    wz = jnp.zeros((256, 512), jnp.float32)
    t_i = jnp.arange(L_READ)
    for mm in range(D_MODEL):
        for d in range(D_INNER):
            wx = wx.at[t_i * 4 + mm, t_i * 8 + d].set(inw[d, mm])
            wz = wz.at[t_i * 4 + mm, t_i * 8 + d].set(inw[D_INNER + d, mm])
    wc = jnp.zeros((512, 512), jnp.float32)
    for k in range(D_CONV):
        tt = jnp.arange(D_CONV - 1 - k, L_READ)
        for d in range(D_INNER):
            wc = wc.at[(tt - (D_CONV - 1 - k)) * 8 + d, tt * 8 + d].set(
                m['conv_w'][d, 0, k])
    cb = jnp.tile(m['conv_b'], (L_READ,))[None, :]
    mdt = m['x_proj_w'][:DT_RANK, :].T @ m['dt_proj_w'].T
    dtb = m['dt_proj_b'][None, :]
    k8 = jnp.zeros((8, 256), jnp.float32)
    d_i = jnp.arange(D_INNER)
    s_i = jnp.arange(D_STATE)
    for s in range(D_STATE):
        k8 = k8.at[d_i, d_i * 32 + s].set(1.0)
    k32 = jnp.zeros((32, 256), jnp.float32)
    for d in range(D_INNER):
        k32 = k32.at[s_i, d * 32 + s_i].set(1.0)
    xpb = m['x_proj_w'][DT_RANK:DT_RANK + D_STATE, :]
    xpc = m['x_proj_w'][DT_RANK + D_STATE:, :]
    wbb = xpb.T @ k32
    wcb = xpc.T @ k32
    af = (-jnp.exp(m['A_log'])).reshape(-1)[None, :]
    df = jnp.tile(m['D'], (L_READ,))[None, :]
    esel = jnp.zeros((L_READ, 512), jnp.float32)
    fsel = jnp.zeros((512, 8), jnp.float32)
    for d in range(D_INNER):
        esel = esel.at[t_i, t_i * 8 + d].set(1.0)
        fsel = fsel.at[t_i * 8 + d, d].set(1.0)
    wo = m['out_proj_w'].T
    wb2 = base_w.T
    bb2 = base_b[None, :]

    blk = lambda i: (i, 0)
    cst = lambda i: (0, 0)
    return pl.pallas_call(
        _mamba_body,
        grid=(_NMB,),
        in_specs=[
            pl.BlockSpec((_MBLK, 256), blk),
            pl.BlockSpec((1, 1, _MBLK), lambda i: (i, 0, 0)),
            pl.BlockSpec((256, 512), cst),
            pl.BlockSpec((256, 512), cst),
            pl.BlockSpec((512, 512), cst),
            pl.BlockSpec((1, 512), cst),
            pl.BlockSpec((8, 8), cst),
            pl.BlockSpec((1, 8), cst),
            pl.BlockSpec((8, 256), cst),
            pl.BlockSpec((8, 256), cst),
            pl.BlockSpec((8, 256), cst),
            pl.BlockSpec((256, 8), cst),
            pl.BlockSpec((1, 256), cst),
            pl.BlockSpec((1, 512), cst),
            pl.BlockSpec((L_READ, 512), cst),
            pl.BlockSpec((512, 8), cst),
            pl.BlockSpec((8, 4), cst),
            pl.BlockSpec((4, D_HID), cst),
            pl.BlockSpec((1, D_HID), cst),
        ],
        out_specs=pl.BlockSpec((_MBLK, D_HID), blk),
        out_shape=jax.ShapeDtypeStruct((N_NODES, D_HID), jnp.float32),
        scratch_shapes=[pltpu.VMEM((_MBLK, 512), jnp.float32)],
    )(rd_flat, rl3, wx, wz, wc, cb, mdt, dtb, wbb, wcb, k8, k8.T,
      af, df, esel, fsel, wo, wb2, bb2)


def _pack_node_w(p):
    wn = jnp.concatenate(
        [p['B1_w'].T, p['A2_w'].T, p['B2_w'].T, p['A3_w'].T], axis=1)
    bn = jnp.concatenate(
        [p['B1_b'], p['A2_b'], p['B2_b'], p['A3_b']])[None, :]
    return wn, bn


def kernel(x, e, edge_index, read_data, read_length, params):
    src = edge_index[0]
    dst = edge_index[1]
    p = params
    gnn = p['gnn']
    zeros_n = jnp.zeros((NPAD, 128), jnp.float32)

    wn1, bn1 = _pack_node_w(gnn[0])
    h, tsrc, tdst = _node_enc(
        x, p['l1n_w'].T, p['l1n_b'][None, :], p['l2n_w'].T, p['l2n_b'][None, :],
        wn1, bn1)
    e_cur, b3e = _edge_enc(
        e, p['l1e_w'].T, p['l1e_b'][None, :], p['l2e_w'].T, p['l2e_b'][None, :],
        gnn[0]['B3_w'].T, gnn[0]['B3_b'][None, :])

    rd_flat = read_data.reshape(N_NODES, L_READ * D_MODEL)
    rl3 = read_length.reshape(_NMB, 1, _MBLK)
    x2 = _mamba(rd_flat, rl3, p['mamba'], p['base_w'], p['base_b'])

    ehat = None
    for li in range(N_LAYERS):
        lp = gnn[li]
        ehat, accf, stats = _sc_pass_f(src, dst, b3e, tsrc, tdst, zeros_n)
        accb = _sc_pass_b(src, dst, ehat, tdst, zeros_n)
        stats2 = stats.reshape(NW, 128)
        bnh = jnp.stack([lp['bn_h_g'], lp['bn_h_b']])
        bne = jnp.stack([lp['bn_e_g'], lp['bn_e_b']])
        if li < N_LAYERS - 1:
            nxt = gnn[li + 1]
            wn, bn = _pack_node_w(nxt)
            h, ss, tsrc, tdst = _node_upd(
                h, lp['A1_w'].T, lp['A1_b'][None, :], accf, accb, stats2,
                bnh, bne, wn, bn)
            e_cur, b3e = _edge_apply(e_cur, ehat, ss, nxt['B3_w'].T,
                                     nxt['B3_b'][None, :])
        else:
            w1s = p['p1_w'][:, :D_HID].T
            w1d = p['p1_w'][:, D_HID:2 * D_HID].T
            ss, pp = _node_fin(
                h, lp['A1_w'].T, lp['A1_b'][None, :], accf, accb, stats2,
                bnh, bne, x2, w1s, w1d)

    pre = _sc_gather_pre(src, dst, pp)
    w1e = p['p1_w'][:, 2 * D_HID:].T
    scores = _predictor(pre, e_cur, ehat, ss, w1e, p['p1_b'][None, :],
                        p['p2_w'].T, p['p2_b'][None, :])
    return scores


# speedup vs baseline: 1.8462x; 1.0234x over previous
import functools

import jax
import jax.numpy as jnp
from jax import lax
from jax.experimental import pallas as pl
from jax.experimental.pallas import tpu as pltpu
from jax.experimental.pallas import tpu_sc as plsc

N_NODES = 10000
N_EDGES = 320000
D_FEAT = 128
D_EDGE = 16
D_INT = 64
D_HID = 64
N_LAYERS = 4
D_SCORE = 64
L_READ = 64
D_MODEL = 4
D_INNER = 8
D_STATE = 32
D_CONV = 4
DT_RANK = 1

NC = 2
NS = 16
NW = NC * NS
EPT = N_EDGES // NW
CH = 80
NCH = EPT // CH
NPAD = 10240
RPT = NPAD // NS

@functools.cache
def _sc_mesh():
    return plsc.VectorSubcoreMesh(core_axis_name="c", subcore_axis_name="s")


def _sigmoid16(x):
    return 1.0 / (1.0 + jnp.exp(-x))


def _scf_body(src_hbm, dst_hbm, b3e_hbm, tsrc_hbm, tdst_hbm, zero_hbm,
              ehat_hbm, acc_hbm, stats_hbm,
              sidx_v, didx_v, b3e_v, srow_v, drow_v, vals_v, stat_v,
              acc_sh, isem1, isem2, sem1, sem2, sem3, wsem):
    cid = lax.axis_index("c")
    sid = lax.axis_index("s")
    wid = sid * NC + cid
    pltpu.sync_copy(zero_hbm.at[pl.ds(sid * RPT, RPT)],
                    acc_sh.at[pl.ds(sid * RPT, RPT)])
    plsc.subcore_barrier()
    ebase = wid * EPT

    def issue_gathers(c):
        base = ebase + c * CH
        pltpu.async_copy(b3e_hbm.at[pl.ds(base, CH)], b3e_v, sem1)
        pltpu.async_copy(tsrc_hbm.at[sidx_v], srow_v, sem2)
        pltpu.async_copy(tdst_hbm.at[didx_v], drow_v, sem3)

    def wait_gathers():
        pltpu.make_async_copy(b3e_hbm.at[pl.ds(ebase, CH)], b3e_v, sem1).wait()
        pltpu.make_async_copy(tsrc_hbm.at[sidx_v], srow_v, sem2).wait()
        pltpu.make_async_copy(tdst_hbm.at[didx_v], drow_v, sem3).wait()

    pltpu.sync_copy(src_hbm.at[pl.ds(ebase, CH)], sidx_v)
    pltpu.sync_copy(dst_hbm.at[pl.ds(ebase, CH)], didx_v)
    issue_gathers(0)

    def chunk(c, stats):
        nbase = ebase + jnp.minimum(c + 1, NCH - 1) * CH
        wait_gathers()
        pltpu.async_copy(src_hbm.at[pl.ds(nbase, CH)], sidx_v, isem1)

        def row(r, st):
            out = []
            for v in range(4):
                j = v * 16
                b3 = b3e_v[r, pl.ds(j, 16)]
                b1 = srow_v[r, pl.ds(j, 16)]
                a2 = srow_v[r, pl.ds(64 + j, 16)]
                b2 = drow_v[r, pl.ds(j, 16)]
                eh = b3 + b1 + b2
                sg = _sigmoid16(eh)
                b3e_v[r, pl.ds(j, 16)] = eh
                vals_v[r, pl.ds(j, 16)] = sg * a2
                vals_v[r, pl.ds(64 + j, 16)] = sg
                out.append(st[2 * v] + eh)
                out.append(st[2 * v + 1] + eh * eh)
            return tuple(out)

        stats = lax.fori_loop(0, CH, row, stats)
        base = ebase + c * CH
        pltpu.async_copy(b3e_v, ehat_hbm.at[pl.ds(base, CH)], wsem)
        pltpu.sync_copy(vals_v, acc_sh.at[didx_v], add=True)
        pltpu.async_copy(dst_hbm.at[pl.ds(nbase, CH)], didx_v, isem2)
        pltpu.make_async_copy(src_hbm.at[pl.ds(ebase, CH)], sidx_v, isem1).wait()
        pltpu.make_async_copy(dst_hbm.at[pl.ds(ebase, CH)], didx_v, isem2).wait()
        pltpu.make_async_copy(b3e_v, ehat_hbm.at[pl.ds(ebase, CH)], wsem).wait()
        issue_gathers(jnp.minimum(c + 1, NCH - 1))
        return stats

    zero16 = jnp.zeros((16,), jnp.float32)
    stats = lax.fori_loop(0, NCH, chunk, tuple(zero16 for _ in range(8)))
    wait_gathers()
    for v in range(4):
        stat_v[v, :] = stats[2 * v]
        stat_v[4 + v, :] = stats[2 * v + 1]
    pltpu.sync_copy(stat_v, stats_hbm.at[wid])
    plsc.subcore_barrier()
    pltpu.sync_copy(acc_sh.at[pl.ds(sid * RPT, RPT)],
                    acc_hbm.at[pl.ds(cid * NPAD + sid * RPT, RPT)])


def _sc_pass_f(src, dst, b3e, tsrc, tdst, zeros_n):
    fn = pl.kernel(
        _scf_body,
        out_type=[
            jax.ShapeDtypeStruct((N_EDGES, D_HID), jnp.float32),
            jax.ShapeDtypeStruct((NC * NPAD, 128), jnp.float32),
            jax.ShapeDtypeStruct((NW, 8, 16), jnp.float32),
        ],
        mesh=_sc_mesh(),
        scratch_types=[
            pltpu.VMEM((CH,), jnp.int32),
            pltpu.VMEM((CH,), jnp.int32),
            pltpu.VMEM((CH, D_HID), jnp.float32),
            pltpu.VMEM((CH, 128), jnp.float32),
            pltpu.VMEM((CH, 128), jnp.float32),
            pltpu.VMEM((CH, 128), jnp.float32),
            pltpu.VMEM((8, 16), jnp.float32),
            pltpu.VMEM_SHARED((NPAD, 128), jnp.float32),
            pltpu.SemaphoreType.DMA,
            pltpu.SemaphoreType.DMA,
            pltpu.SemaphoreType.DMA,
            pltpu.SemaphoreType.DMA,
            pltpu.SemaphoreType.DMA,
            pltpu.SemaphoreType.DMA,
        ],
    )
    return fn(src, dst, b3e, tsrc, tdst, zeros_n)


def _scb_body(src_hbm, dst_hbm, ehat_hbm, tdst_hbm, zero_hbm,
              acc_hbm,
              sidx_v, didx_v, ehat_v, arow_v,
              acc_sh, isem1, isem2, sem1, sem2):
    cid = lax.axis_index("c")
    sid = lax.axis_index("s")
    wid = sid * NC + cid
    pltpu.sync_copy(zero_hbm.at[pl.ds(sid * RPT, RPT)],
                    acc_sh.at[pl.ds(sid * RPT, RPT)])
    plsc.subcore_barrier()
    ebase = wid * EPT

    def issue_gathers(c):
        base = ebase + c * CH
        pltpu.async_copy(ehat_hbm.at[pl.ds(base, CH)], ehat_v, sem1)
        pltpu.async_copy(tdst_hbm.at[didx_v], arow_v, sem2)

    def wait_gathers():
        pltpu.make_async_copy(ehat_hbm.at[pl.ds(ebase, CH)], ehat_v, sem1).wait()
        pltpu.make_async_copy(tdst_hbm.at[didx_v], arow_v, sem2).wait()

    pltpu.sync_copy(src_hbm.at[pl.ds(ebase, CH)], sidx_v)
    pltpu.sync_copy(dst_hbm.at[pl.ds(ebase, CH)], didx_v)
    issue_gathers(0)

    def chunk(c, carry):
        nbase = ebase + jnp.minimum(c + 1, NCH - 1) * CH
        wait_gathers()
        pltpu.async_copy(dst_hbm.at[pl.ds(nbase, CH)], didx_v, isem2)

        def row(r, cr):
            for v in range(4):
                j = v * 16
                eh = ehat_v[r, pl.ds(j, 16)]
                a3 = arow_v[r, pl.ds(64 + j, 16)]
                sg = _sigmoid16(eh)
                arow_v[r, pl.ds(j, 16)] = sg * a3
                arow_v[r, pl.ds(64 + j, 16)] = sg
            return cr

        lax.fori_loop(0, CH, row, 0)
        pltpu.sync_copy(arow_v, acc_sh.at[sidx_v], add=True)
        pltpu.async_copy(src_hbm.at[pl.ds(nbase, CH)], sidx_v, isem1)
        pltpu.make_async_copy(src_hbm.at[pl.ds(ebase, CH)], sidx_v, isem1).wait()
        pltpu.make_async_copy(dst_hbm.at[pl.ds(ebase, CH)], didx_v, isem2).wait()
        issue_gathers(jnp.minimum(c + 1, NCH - 1))
        return carry

    lax.fori_loop(0, NCH, chunk, 0)
    wait_gathers()
    plsc.subcore_barrier()
    pltpu.sync_copy(acc_sh.at[pl.ds(sid * RPT, RPT)],
                    acc_hbm.at[pl.ds(cid * NPAD + sid * RPT, RPT)])


def _sc_pass_b(src, dst, ehat, tdst, zeros_n):
    fn = pl.kernel(
        _scb_body,
        out_type=[
            jax.ShapeDtypeStruct((NC * NPAD, 128), jnp.float32),
        ],
        mesh=_sc_mesh(),
        scratch_types=[
            pltpu.VMEM((CH,), jnp.int32),
            pltpu.VMEM((CH,), jnp.int32),
            pltpu.VMEM((CH, D_HID), jnp.float32),
            pltpu.VMEM((CH, 128), jnp.float32),
            pltpu.VMEM_SHARED((NPAD, 128), jnp.float32),
            pltpu.SemaphoreType.DMA,
            pltpu.SemaphoreType.DMA,
            pltpu.SemaphoreType.DMA,
            pltpu.SemaphoreType.DMA,
        ],
    )
    return fn(src, dst, ehat, tdst, zeros_n)[0]


def _scg_body(src_hbm, dst_hbm, pp_hbm,
              pre_hbm,
              sidx_v, didx_v, ps_v, pd_v, out_v, isem1, isem2,
              sem1, sem2, wsem):
    cid = lax.axis_index("c")
    sid = lax.axis_index("s")
    wid = sid * NC + cid
    ebase = wid * EPT

    def issue_gathers():
        pltpu.async_copy(pp_hbm.at[sidx_v], ps_v, sem1)
        pltpu.async_copy(pp_hbm.at[didx_v], pd_v, sem2)

    def wait_gathers():
        pltpu.make_async_copy(pp_hbm.at[sidx_v], ps_v, sem1).wait()
        pltpu.make_async_copy(pp_hbm.at[didx_v], pd_v, sem2).wait()

    pltpu.sync_copy(src_hbm.at[pl.ds(ebase, CH)], sidx_v)
    pltpu.sync_copy(dst_hbm.at[pl.ds(ebase, CH)], didx_v)
    issue_gathers()

    def chunk(c, carry):
        nbase = ebase + jnp.minimum(c + 1, NCH - 1) * CH
        wait_gathers()
        pltpu.async_copy(src_hbm.at[pl.ds(nbase, CH)], sidx_v, isem1)
        pltpu.async_copy(dst_hbm.at[pl.ds(nbase, CH)], didx_v, isem2)

        def row(r, cr):
            for v in range(4):
                j = v * 16
                out_v[r, pl.ds(j, 16)] = (ps_v[r, pl.ds(j, 16)]
                                          + pd_v[r, pl.ds(64 + j, 16)])
            return cr

        lax.fori_loop(0, CH, row, 0)
        base = ebase + c * CH
        pltpu.async_copy(out_v, pre_hbm.at[pl.ds(base, CH)], wsem)
        pltpu.make_async_copy(src_hbm.at[pl.ds(ebase, CH)], sidx_v, isem1).wait()
        pltpu.make_async_copy(dst_hbm.at[pl.ds(ebase, CH)], didx_v, isem2).wait()
        issue_gathers()
        pltpu.make_async_copy(out_v, pre_hbm.at[pl.ds(ebase, CH)], wsem).wait()
        return carry

    lax.fori_loop(0, NCH, chunk, 0)
    wait_gathers()


def _sc_gather_pre(src, dst, pp):
    fn = pl.kernel(
        _scg_body,
        out_type=[jax.ShapeDtypeStruct((N_EDGES, D_HID), jnp.float32)],
        mesh=_sc_mesh(),
        scratch_types=[
            pltpu.VMEM((CH,), jnp.int32),
            pltpu.VMEM((CH,), jnp.int32),
            pltpu.VMEM((CH, 128), jnp.float32),
            pltpu.VMEM((CH, 128), jnp.float32),
            pltpu.VMEM((CH, D_HID), jnp.float32),
            pltpu.SemaphoreType.DMA,
            pltpu.SemaphoreType.DMA,
            pltpu.SemaphoreType.DMA,
            pltpu.SemaphoreType.DMA,
            pltpu.SemaphoreType.DMA,
        ],
    )
    return fn(src, dst, pp)[0]


def _node_enc_body(x_ref, w1_ref, b1_ref, w2_ref, b2_ref, wn_ref, bn_ref,
                   h_ref, tsrc_ref, tdst_ref):
    h = jnp.maximum(x_ref[...] @ w1_ref[...] + b1_ref[...], 0.0)
    h = h @ w2_ref[...] + b2_ref[...]
    h_ref[...] = h
    proj = h @ wn_ref[...] + bn_ref[...]
    tsrc_ref[...] = proj[:, :128]
    tdst_ref[...] = proj[:, 128:256]


def _node_enc(x, w1, b1, w2, b2, wn, bn):
    return pl.pallas_call(
        _node_enc_body,
        out_shape=[
            jax.ShapeDtypeStruct((N_NODES, D_HID), jnp.float32),
            jax.ShapeDtypeStruct((N_NODES, 128), jnp.float32),
            jax.ShapeDtypeStruct((N_NODES, 128), jnp.float32),
        ],
    )(x, w1, b1, w2, b2, wn, bn)


_EBLK = 6400
_NEB = N_EDGES // _EBLK


def _edge_enc_body(e_ref, w1_ref, b1_ref, w2_ref, b2_ref, w3_ref, b3_ref,
                   e0_ref, b3e_ref):
    e = jnp.maximum(e_ref[...] @ w1_ref[...] + b1_ref[...], 0.0)
    e = e @ w2_ref[...] + b2_ref[...]
    e0_ref[...] = e
    b3e_ref[...] = e @ w3_ref[...] + b3_ref[...]


def _edge_enc(e, w1, b1, w2, b2, w3, b3):
    blk = lambda i: (i, 0)
    cst = lambda i: (0, 0)
    return pl.pallas_call(
        _edge_enc_body,
        grid=(_NEB,),
        in_specs=[
            pl.BlockSpec((_EBLK, D_EDGE), blk),
            pl.BlockSpec((D_EDGE, D_INT), cst),
            pl.BlockSpec((1, D_INT), cst),
            pl.BlockSpec((D_INT, D_HID), cst),
            pl.BlockSpec((1, D_HID), cst),
            pl.BlockSpec((D_HID, D_HID), cst),
            pl.BlockSpec((1, D_HID), cst),
        ],
        out_specs=[
            pl.BlockSpec((_EBLK, D_HID), blk),
            pl.BlockSpec((_EBLK, D_HID), blk),
        ],
        out_shape=[
            jax.ShapeDtypeStruct((N_EDGES, D_HID), jnp.float32),
            jax.ShapeDtypeStruct((N_EDGES, D_HID), jnp.float32),
        ],
    )(e, w1, b1, w2, b2, w3, b3)


def _node_upd_body(h_ref, a1w_ref, a1b_ref, accf_ref, accb_ref, stats_ref,
                   bnh_ref, bne_ref, wn_ref, bn_ref,
                   h2_ref, ss_ref, tsrc_ref, tdst_ref):
    h = h_ref[...]
    a1h = h @ a1w_ref[...] + a1b_ref[...]
    accf = accf_ref[...]
    accb = accb_ref[...]
    num_f = accf[:N_NODES, :64] + accf[NPAD:NPAD + N_NODES, :64]
    den_f = accf[:N_NODES, 64:] + accf[NPAD:NPAD + N_NODES, 64:]
    num_b = accb[:N_NODES, :64] + accb[NPAD:NPAD + N_NODES, :64]
    den_b = accb[:N_NODES, 64:] + accb[NPAD:NPAD + N_NODES, 64:]
    tmp = a1h + num_f / (den_f + 1e-6) + num_b / (den_b + 1e-6)
    mu = jnp.mean(tmp, axis=0, keepdims=True)
    var = jnp.mean((tmp - mu) ** 2, axis=0, keepdims=True)
    bnh = bnh_ref[...]
    hn = (tmp - mu) / jnp.sqrt(var + 1e-5) * bnh[0:1, :] + bnh[1:2, :]
    h2 = h + jnp.maximum(hn, 0.0)
    h2_ref[...] = h2
    st = jnp.sum(stats_ref[...], axis=0)
    mu_e = st[:64] / N_EDGES
    var_e = st[64:] / N_EDGES - mu_e * mu_e
    bne = bne_ref[...]
    scale = bne[0, :] / jnp.sqrt(var_e + 1e-5)
    shift = bne[1, :] - mu_e * scale
    ss_ref[...] = jnp.concatenate(
        [scale[None, :], shift[None, :], jnp.zeros((6, D_HID), jnp.float32)],
        axis=0)
    proj = h2 @ wn_ref[...] + bn_ref[...]
    tsrc_ref[...] = proj[:, :128]
    tdst_ref[...] = proj[:, 128:256]


def _node_upd(h, a1w, a1b, accf, accb, stats, bnh, bne, wn, bn):
    return pl.pallas_call(
        _node_upd_body,
        out_shape=[
            jax.ShapeDtypeStruct((N_NODES, D_HID), jnp.float32),
            jax.ShapeDtypeStruct((8, D_HID), jnp.float32),
            jax.ShapeDtypeStruct((N_NODES, 128), jnp.float32),
            jax.ShapeDtypeStruct((N_NODES, 128), jnp.float32),
        ],
    )(h, a1w, a1b, accf, accb, stats, bnh, bne, wn, bn)


def _node_fin_body(h_ref, a1w_ref, a1b_ref, accf_ref, accb_ref, stats_ref,
                   bnh_ref, bne_ref, x2_ref, ws_ref, wd_ref,
                   ss_ref, pp_ref):
    h = h_ref[...]
    a1h = h @ a1w_ref[...] + a1b_ref[...]
    accf = accf_ref[...]
    accb = accb_ref[...]
    num_f = accf[:N_NODES, :64] + accf[NPAD:NPAD + N_NODES, :64]
    den_f = accf[:N_NODES, 64:] + accf[NPAD:NPAD + N_NODES, 64:]
    num_b = accb[:N_NODES, :64] + accb[NPAD:NPAD + N_NODES, :64]
    den_b = accb[:N_NODES, 64:] + accb[NPAD:NPAD + N_NODES, 64:]
    tmp = a1h + num_f / (den_f + 1e-6) + num_b / (den_b + 1e-6)
    mu = jnp.mean(tmp, axis=0, keepdims=True)
    var = jnp.mean((tmp - mu) ** 2, axis=0, keepdims=True)
    bnh = bnh_ref[...]
    hn = (tmp - mu) / jnp.sqrt(var + 1e-5) * bnh[0:1, :] + bnh[1:2, :]
    hf = h + jnp.maximum(hn, 0.0) + x2_ref[...]
    st = jnp.sum(stats_ref[...], axis=0)
    mu_e = st[:64] / N_EDGES
    var_e = st[64:] / N_EDGES - mu_e * mu_e
    bne = bne_ref[...]
    scale = bne[0, :] / jnp.sqrt(var_e + 1e-5)
    shift = bne[1, :] - mu_e * scale
    ss_ref[...] = jnp.concatenate(
        [scale[None, :], shift[None, :], jnp.zeros((6, D_HID), jnp.float32)],
        axis=0)
    pp_ref[...] = jnp.concatenate([hf @ ws_ref[...], hf @ wd_ref[...]],
                                  axis=1)


def _node_fin(h, a1w, a1b, accf, accb, stats, bnh, bne, x2, ws, wd):
    return pl.pallas_call(
        _node_fin_body,
        out_shape=[
            jax.ShapeDtypeStruct((8, D_HID), jnp.float32),
            jax.ShapeDtypeStruct((N_NODES, 128), jnp.float32),
        ],
    )(h, a1w, a1b, accf, accb, stats, bnh, bne, x2, ws, wd)


def _edge_apply_body(e_ref, ehat_ref, ss_ref, w_ref, b_ref,
                     e2_ref, b3e_ref):
    ss = ss_ref[...]
    en = e_ref[...] + jnp.maximum(ehat_ref[...] * ss[0:1, :] + ss[1:2, :], 0.0)
    e2_ref[...] = en
    b3e_ref[...] = en @ w_ref[...] + b_ref[...]


def _edge_apply(e, ehat, ss, w, b):
    blk = lambda i: (i, 0)
    cst = lambda i: (0, 0)
    return pl.pallas_call(
        _edge_apply_body,
        grid=(_NEB,),
        in_specs=[
            pl.BlockSpec((_EBLK, D_HID), blk),
            pl.BlockSpec((_EBLK, D_HID), blk),
            pl.BlockSpec((8, D_HID), cst),
            pl.BlockSpec((D_HID, D_HID), cst),
            pl.BlockSpec((1, D_HID), cst),
        ],
        out_specs=[
            pl.BlockSpec((_EBLK, D_HID), blk),
            pl.BlockSpec((_EBLK, D_HID), blk),
        ],
        out_shape=[
            jax.ShapeDtypeStruct((N_EDGES, D_HID), jnp.float32),
            jax.ShapeDtypeStruct((N_EDGES, D_HID), jnp.float32),
        ],
    )(e, ehat, ss, w, b)


def _pred_body(pre_ref, e_ref, ehat_ref, ss_ref, w1e_ref, b1_ref,
               w2_ref, b2_ref, out_ref):
    ss = ss_ref[...]
    e4 = e_ref[...] + jnp.maximum(ehat_ref[...] * ss[0:1, :] + ss[1:2, :], 0.0)
    hcat = pre_ref[...] + e4 @ w1e_ref[...] + b1_ref[...]
    hcat = jnp.maximum(hcat, 0.0)
    out_ref[...] = hcat @ w2_ref[...] + b2_ref[...]


def _predictor(pre, e3, ehat4, ss, w1e, b1, w2, b2):
    blk = lambda i: (i, 0)
    cst = lambda i: (0, 0)
    return pl.pallas_call(
        _pred_body,
        grid=(_NEB,),
        in_specs=[
            pl.BlockSpec((_EBLK, D_HID), blk),
            pl.BlockSpec((_EBLK, D_HID), blk),
            pl.BlockSpec((_EBLK, D_HID), blk),
            pl.BlockSpec((8, D_HID), cst),
            pl.BlockSpec((D_HID, D_SCORE), cst),
            pl.BlockSpec((1, D_SCORE), cst),
            pl.BlockSpec((D_SCORE, 1), cst),
            pl.BlockSpec((1, 1), cst),
        ],
        out_specs=pl.BlockSpec((_EBLK, 1), blk),
        out_shape=jax.ShapeDtypeStruct((N_EDGES, 1), jnp.float32),
    )(pre, e3, ehat4, ss, w1e, b1, w2, b2)


_MBLK = 400
_NMB = N_NODES // _MBLK


def _mamba_body(rd_ref, rl_ref, wx_ref, wz_ref, wc_ref, cb_ref,
                mdt_ref, dtb_ref, wbb_ref, wcb_ref, k8_ref, k8t_ref,
                af_ref, df_ref, esel_ref, fsel_ref,
                wo_ref, wb2_ref, bb2_ref, x2_ref, ys_ref):
    rd = rd_ref[...]
    xm = rd @ wx_ref[...]
    z = rd @ wz_ref[...]
    xc = xm @ wc_ref[...] + cb_ref[...]
    xc = xc * _sigmoid16(xc)
    k8 = k8_ref[...]
    af = af_ref[...]
    h = jnp.zeros((_MBLK, 256), jnp.float32)
    for t in range(L_READ):
        xct = xc[:, t * 8:(t + 1) * 8]
        dpre = xct @ mdt_ref[...] + dtb_ref[...]
        dt = jnp.maximum(dpre, 0.0) + jnp.log1p(jnp.exp(-jnp.abs(dpre)))
        d_bc = dt @ k8
        b_bc = xct @ wbb_ref[...]
        c_bc = xct @ wcb_ref[...]
        u_bc = xct @ k8
        dA = jnp.exp(d_bc * af)
        h = dA * h + d_bc * b_bc * u_bc
        yt = (h * c_bc) @ k8t_ref[...]
        ys_ref[:, t * 8:(t + 1) * 8] = yt
    y = ys_ref[...] + xc * df_ref[...]
    y = y * (z * _sigmoid16(z))
    idx = jnp.clip(rl_ref[0, 0, :] - 1, 0, L_READ - 1)
    tmask = (jax.lax.broadcasted_iota(jnp.int32, (_MBLK, L_READ), 1)
             == idx[:, None]).astype(jnp.float32)
    msel = tmask @ esel_ref[...]
    ylast = (y * msel) @ fsel_ref[...]
    out4 = ylast @ wo_ref[...]
    x2_ref[...] = out4 @ wb2_ref[...] + bb2_ref[...]


def _mamba(rd_flat, rl3, m, base_w, base_b):
    inw = m['in_proj_w']
    wx = jnp.zeros((256, 512), jnp.float32)
    wz = jnp.zeros((256, 512), jnp.float32)
    t_i = jnp.arange(L_READ)
    for mm in range(D_MODEL):
        for d in range(D_INNER):
            wx = wx.at[t_i * 4 + mm, t_i * 8 + d].set(inw[d, mm])
            wz = wz.at[t_i * 4 + mm, t_i * 8 + d].set(inw[D_INNER + d, mm])
    wc = jnp.zeros((512, 512), jnp.float32)
    for k in range(D_CONV):
        tt = jnp.arange(D_CONV - 1 - k, L_READ)
        for d in range(D_INNER):
            wc = wc.at[(tt - (D_CONV - 1 - k)) * 8 + d, tt * 8 + d].set(
                m['conv_w'][d, 0, k])
    cb = jnp.tile(m['conv_b'], (L_READ,))[None, :]
    mdt = m['x_proj_w'][:DT_RANK, :].T @ m['dt_proj_w'].T
    dtb = m['dt_proj_b'][None, :]
    k8 = jnp.zeros((8, 256), jnp.float32)
    d_i = jnp.arange(D_INNER)
    s_i = jnp.arange(D_STATE)
    for s in range(D_STATE):
        k8 = k8.at[d_i, d_i * 32 + s].set(1.0)
    k32 = jnp.zeros((32, 256), jnp.float32)
    for d in range(D_INNER):
        k32 = k32.at[s_i, d * 32 + s_i].set(1.0)
    xpb = m['x_proj_w'][DT_RANK:DT_RANK + D_STATE, :]
    xpc = m['x_proj_w'][DT_RANK + D_STATE:, :]
    wbb = xpb.T @ k32
    wcb = xpc.T @ k32
    af = (-jnp.exp(m['A_log'])).reshape(-1)[None, :]
    df = jnp.tile(m['D'], (L_READ,))[None, :]
    esel = jnp.zeros((L_READ, 512), jnp.float32)
    fsel = jnp.zeros((512, 8), jnp.float32)
    for d in range(D_INNER):
        esel = esel.at[t_i, t_i * 8 + d].set(1.0)
        fsel = fsel.at[t_i * 8 + d, d].set(1.0)
    wo = m['out_proj_w'].T
    wb2 = base_w.T
    bb2 = base_b[None, :]

    blk = lambda i: (i, 0)
    cst = lambda i: (0, 0)
    return pl.pallas_call(
        _mamba_body,
        grid=(_NMB,),
        in_specs=[
            pl.BlockSpec((_MBLK, 256), blk),
            pl.BlockSpec((1, 1, _MBLK), lambda i: (i, 0, 0)),
            pl.BlockSpec((256, 512), cst),
            pl.BlockSpec((256, 512), cst),
            pl.BlockSpec((512, 512), cst),
            pl.BlockSpec((1, 512), cst),
            pl.BlockSpec((8, 8), cst),
            pl.BlockSpec((1, 8), cst),
            pl.BlockSpec((8, 256), cst),
            pl.BlockSpec((8, 256), cst),
            pl.BlockSpec((8, 256), cst),
            pl.BlockSpec((256, 8), cst),
            pl.BlockSpec((1, 256), cst),
            pl.BlockSpec((1, 512), cst),
            pl.BlockSpec((L_READ, 512), cst),
            pl.BlockSpec((512, 8), cst),
            pl.BlockSpec((8, 4), cst),
            pl.BlockSpec((4, D_HID), cst),
            pl.BlockSpec((1, D_HID), cst),
        ],
        out_specs=pl.BlockSpec((_MBLK, D_HID), blk),
        out_shape=jax.ShapeDtypeStruct((N_NODES, D_HID), jnp.float32),
        scratch_shapes=[pltpu.VMEM((_MBLK, 512), jnp.float32)],
    )(rd_flat, rl3, wx, wz, wc, cb, mdt, dtb, wbb, wcb, k8, k8.T,
      af, df, esel, fsel, wo, wb2, bb2)


def _pack_node_w(p):
    wn = jnp.concatenate(
        [p['B1_w'].T, p['A2_w'].T, p['B2_w'].T, p['A3_w'].T], axis=1)
    bn = jnp.concatenate(
        [p['B1_b'], p['A2_b'], p['B2_b'], p['A3_b']])[None, :]
    return wn, bn


def kernel(x, e, edge_index, read_data, read_length, params):
    src = edge_index[0]
    dst = edge_index[1]
    p = params
    gnn = p['gnn']
    zeros_n = jnp.zeros((NPAD, 128), jnp.float32)

    wn1, bn1 = _pack_node_w(gnn[0])
    h, tsrc, tdst = _node_enc(
        x, p['l1n_w'].T, p['l1n_b'][None, :], p['l2n_w'].T, p['l2n_b'][None, :],
        wn1, bn1)
    e_cur, b3e = _edge_enc(
        e, p['l1e_w'].T, p['l1e_b'][None, :], p['l2e_w'].T, p['l2e_b'][None, :],
        gnn[0]['B3_w'].T, gnn[0]['B3_b'][None, :])

    rd_flat = read_data.reshape(N_NODES, L_READ * D_MODEL)
    rl3 = read_length.reshape(_NMB, 1, _MBLK)
    x2 = _mamba(rd_flat, rl3, p['mamba'], p['base_w'], p['base_b'])

    ehat = None
    for li in range(N_LAYERS):
        lp = gnn[li]
        ehat, accf, stats = _sc_pass_f(src, dst, b3e, tsrc, tdst, zeros_n)
        accb = _sc_pass_b(src, dst, ehat, tdst, zeros_n)
        stats2 = stats.reshape(NW, 128)
        bnh = jnp.stack([lp['bn_h_g'], lp['bn_h_b']])
        bne = jnp.stack([lp['bn_e_g'], lp['bn_e_b']])
        if li < N_LAYERS - 1:
            nxt = gnn[li + 1]
            wn, bn = _pack_node_w(nxt)
            h, ss, tsrc, tdst = _node_upd(
                h, lp['A1_w'].T, lp['A1_b'][None, :], accf, accb, stats2,
                bnh, bne, wn, bn)
            e_cur, b3e = _edge_apply(e_cur, ehat, ss, nxt['B3_w'].T,
                                     nxt['B3_b'][None, :])
        else:
            w1s = p['p1_w'][:, :D_HID].T
            w1d = p['p1_w'][:, D_HID:2 * D_HID].T
            ss, pp = _node_fin(
                h, lp['A1_w'].T, lp['A1_b'][None, :], accf, accb, stats2,
                bnh, bne, x2, w1s, w1d)

    pre = _sc_gather_pre(src, dst, pp)
    w1e = p['p1_w'][:, 2 * D_HID:].T
    scores = _predictor(pre, e_cur, ehat, ss, w1e, p['p1_b'][None, :],
                        p['p2_w'].T, p['p2_b'][None, :])
    return scores
